# Initial kernel scaffold; baseline (speedup 1.0000x reference)
import jax
import jax.numpy as jnp
from jax import lax
from jax.experimental import pallas as pl
from jax.experimental.pallas import tpu as pltpu
from jax.experimental.pallas import tpu_sc as plsc

_B = 8
_SEQ = 2048
_DLM = 1024
_H = 128
_R = 3
_NL = 2
_C = 8
_N = _B * _SEQ
_E = 524288
_NT = 16
_Q = _N // 4
_ACC_ROWS = _R * _Q + 512
_PAD_BASE = _R * _Q
_EPT = _E // _NT
_HALF = _EPT // 2
_SCAN = 4096
_CH = 128


def _sc_segment_mean(x, src, dst, etyp):
  mesh = plsc.VectorSubcoreMesh(core_axis_name="c", subcore_axis_name="s")

  def body(x_hbm, src_hbm, dst_hbm, typ_hbm, s_hbm,
           acc, cntacc, stage_s, stage_d, stage_t, msrc, mloc, loc2d,
           gbuf, ones, cbuf, gsem, ssem, csem):
    c = lax.axis_index("c")
    w = lax.axis_index("s")
    e0 = w * _EPT
    lane = lax.iota(jnp.int32, 16)

    def _ones_row(i, _):
      ones[i, :] = jnp.ones((16,), jnp.float32)
      return 0
    lax.fori_loop(0, _CH, _ones_row, 0)

    def pass_body(p, _):
      base = (c * 2 + p) * _Q

      def _z_row(i, _):
        for k in range(8):
          gbuf[0, i, pl.ds(k * 16, 16)] = jnp.zeros((16,), jnp.float32)
        cbuf[i, :] = jnp.zeros((16,), jnp.float32)
        return 0
      lax.fori_loop(0, _CH, _z_row, 0)

      z0 = w * (_ACC_ROWS // _NT)
      for t in range(6):
        pltpu.sync_copy(gbuf.at[0], acc.at[pl.ds(z0 + t * _CH, _CH)])
        pltpu.sync_copy(cbuf, cntacc.at[pl.ds(z0 + t * _CH, _CH)])
      pltpu.sync_copy(gbuf.at[0, pl.ds(0, 32)], acc.at[pl.ds(z0 + 768, 32)])
      pltpu.sync_copy(cbuf.at[pl.ds(0, 32)], cntacc.at[pl.ds(z0 + 768, 32)])
      plsc.subcore_barrier()

      def half_body(h, _):
        eh = e0 + h * _HALF

        def _pad_fill(i, _):
          flat = i * 16 + lane
          msrc[pl.ds(i * 16, 16)] = flat & (_N - 1)
          mloc[pl.ds(i * 16, 16)] = _PAD_BASE + (flat & 511)
          return 0
        lax.fori_loop(0, _HALF // 16, _pad_fill, 0)

        def blk_body(blk, ptr):
          eb = eh + blk * _SCAN
          pltpu.sync_copy(src_hbm.at[pl.ds(eb, _SCAN)], stage_s)
          pltpu.sync_copy(dst_hbm.at[pl.ds(eb, _SCAN)], stage_d)
          pltpu.sync_copy(typ_hbm.at[pl.ds(eb, _SCAN)], stage_t)

          def scan_body(i, ptr):
            s = stage_s[pl.ds(i * 16, 16)]
            d = stage_d[pl.ds(i * 16, 16)]
            t = stage_t[pl.ds(i * 16, 16)]
            m = jnp.logical_and(d >= base, d < base + _Q)
            loc = t * _Q + (d - base)
            plsc.store_compressed(msrc.at[pl.ds(ptr, 16)], s, mask=m)
            plsc.store_compressed(mloc.at[pl.ds(ptr, 16)], loc, mask=m)
            return ptr + jnp.sum(m.astype(jnp.int32))

          return lax.fori_loop(0, _SCAN // 16, scan_body, ptr)

        ptr = lax.fori_loop(0, _HALF // _SCAN, blk_body, jnp.int32(0))
        nch = (ptr + (_CH - 1)) // _CH

        def _repack(i, _):
          row = i // 8
          col = (i - row * 8) * 16
          loc2d[row, pl.ds(col, 16)] = mloc[pl.ds(i * 16, 16)]
          return 0
        lax.fori_loop(0, nch * 8, _repack, 0)

        def _start_gather(j):
          b = j & 1
          pltpu.async_copy(x_hbm.at[msrc.at[pl.ds(j * _CH, _CH)]],
                           gbuf.at[b], gsem.at[b])

        @pl.when(nch > 0)
        def _():
          _start_gather(0)

        @pl.when(nch > 1)
        def _():
          _start_gather(1)

        def chunk_body(j, _):
          b = j & 1
          pltpu.make_async_copy(x_hbm.at[msrc.at[pl.ds(j * _CH, _CH)]],
                                gbuf.at[b], gsem.at[b]).wait()
          pltpu.async_copy(gbuf.at[b], acc.at[loc2d.at[j]], ssem, add=True)
          pltpu.async_copy(ones, cntacc.at[loc2d.at[j]], csem, add=True)
          pltpu.make_async_copy(gbuf.at[b], acc.at[loc2d.at[j]], ssem).wait()
          pltpu.make_async_copy(ones, cntacc.at[loc2d.at[j]], csem).wait()

          @pl.when(j + 2 < nch)
          def _():
            _start_gather(j + 2)
          return 0

        lax.fori_loop(0, nch, chunk_body, 0)
        return 0

      lax.fori_loop(0, 2, half_body, 0)
      plsc.subcore_barrier()

      def norm_chunk(t, _):
        l0 = w * 768 + t * _CH
        pltpu.sync_copy(acc.at[pl.ds(l0, _CH)], gbuf.at[0])
        pltpu.sync_copy(cntacc.at[pl.ds(l0, _CH)], cbuf)

        def norm_row(i, _):
          cv = cbuf[i, 0]
          inv = 1.0 / jnp.maximum(cv, 1.0)
          for k in range(8):
            gbuf[0, i, pl.ds(k * 16, 16)] = gbuf[0, i, pl.ds(k * 16, 16)] * inv
          return 0
        lax.fori_loop(0, _CH, norm_row, 0)

        rel = l0 // _Q
        soff = rel * _N + base + (l0 - rel * _Q)
        pltpu.sync_copy(gbuf.at[0], s_hbm.at[pl.ds(soff, _CH)])
        return 0

      lax.fori_loop(0, 6, norm_chunk, 0)
      plsc.subcore_barrier()
      return 0

    lax.fori_loop(0, 2, pass_body, 0)

  fn = pl.kernel(
      body,
      out_type=jax.ShapeDtypeStruct((_R * _N, _H), jnp.float32),
      mesh=mesh,
      scratch_types=[
          pltpu.VMEM_SHARED((_ACC_ROWS, _H), jnp.float32),
          pltpu.VMEM_SHARED((_ACC_ROWS, 16), jnp.float32),
          pltpu.VMEM((_SCAN,), jnp.int32),
          pltpu.VMEM((_SCAN,), jnp.int32),
          pltpu.VMEM((_SCAN,), jnp.int32),
          pltpu.VMEM((_HALF,), jnp.int32),
          pltpu.VMEM((_HALF,), jnp.int32),
          pltpu.VMEM((_HALF // _CH, _CH), jnp.int32),
          pltpu.VMEM((2, _CH, _H), jnp.float32),
          pltpu.VMEM((_CH, 16), jnp.float32),
          pltpu.VMEM((_CH, 16), jnp.float32),
          pltpu.SemaphoreType.DMA((2,)),
          pltpu.SemaphoreType.DMA,
          pltpu.SemaphoreType.DMA,
      ],
  )
  return fn(x, src, dst, etyp)


def _lm_head(xflat, lm_W, lm_b, ln_g, ln_b):
  blk = 256
  grid = _N // blk

  def body(x_ref, w_ref, b_ref, g_ref, lb_ref, o_ref):
    h = jnp.dot(x_ref[...], w_ref[...], preferred_element_type=jnp.float32)
    h = jnp.maximum(h + b_ref[...], 0.0)
    mu = jnp.mean(h, axis=-1, keepdims=True)
    var = jnp.mean((h - mu) ** 2, axis=-1, keepdims=True)
    o_ref[...] = (h - mu) * lax.rsqrt(var + 1e-5) * g_ref[...] + lb_ref[...]

  return pl.pallas_call(
      body,
      grid=(grid,),
      in_specs=[
          pl.BlockSpec((blk, _DLM), lambda i: (i, 0)),
          pl.BlockSpec((_DLM, _H), lambda i: (0, 0)),
          pl.BlockSpec((1, _H), lambda i: (0, 0)),
          pl.BlockSpec((1, _H), lambda i: (0, 0)),
          pl.BlockSpec((1, _H), lambda i: (0, 0)),
      ],
      out_specs=pl.BlockSpec((blk, _H), lambda i: (i, 0)),
      out_shape=jax.ShapeDtypeStruct((_N, _H), jnp.float32),
  )(xflat, lm_W, lm_b.reshape(1, _H), ln_g.reshape(1, _H), ln_b.reshape(1, _H))


def _rgcn_dense(x, s3, root, wr, bias):
  blk = 256
  grid = _N // blk

  def body(x_ref, s_ref, r_ref, w_ref, b_ref, o_ref):
    o = jnp.dot(x_ref[...], r_ref[...], preferred_element_type=jnp.float32)
    o = o + b_ref[...]
    for r in range(_R):
      o = o + jnp.dot(s_ref[r], w_ref[r], preferred_element_type=jnp.float32)
    o_ref[...] = jnp.maximum(o, 0.0)

  return pl.pallas_call(
      body,
      grid=(grid,),
      in_specs=[
          pl.BlockSpec((blk, _H), lambda i: (i, 0)),
          pl.BlockSpec((_R, blk, _H), lambda i: (0, i, 0)),
          pl.BlockSpec((_H, _H), lambda i: (0, 0)),
          pl.BlockSpec((_R, _H, _H), lambda i: (0, 0, 0)),
          pl.BlockSpec((1, _H), lambda i: (0, 0)),
      ],
      out_specs=pl.BlockSpec((blk, _H), lambda i: (i, 0)),
      out_shape=jax.ShapeDtypeStruct((_N, _H), jnp.float32),
  )(x, s3, root, wr, bias.reshape(1, _H))


def _cls_head(lm, g, wpad, bpad, mask2d, labels2d):
  blk = 512
  grid = _N // blk

  def body(lm_ref, g_ref, w_ref, b_ref, m_ref, l_ref, lo_ref, loss_ref, acc):
    i = pl.program_id(0)
    logits = (jnp.dot(lm_ref[...], w_ref[0:_H], preferred_element_type=jnp.float32)
              + jnp.dot(g_ref[...], w_ref[_H:2 * _H], preferred_element_type=jnp.float32)
              + b_ref[...])
    lo_ref[...] = logits
    mx = jnp.max(logits, axis=-1, keepdims=True)
    lse = jnp.log(jnp.sum(jnp.exp(logits - mx), axis=-1, keepdims=True)) + mx
    lab = l_ref[0, :]
    cols = lax.broadcasted_iota(jnp.int32, (blk, _H), 1)
    pick = jnp.sum(jnp.where(cols == lab[:, None], logits, 0.0), axis=-1,
                   keepdims=True)
    active = (m_ref[0, :] == 1).astype(jnp.float32)[:, None]
    bsum = jnp.sum((lse - pick) * active)
    bcnt = jnp.sum(active)
    prev_s = jnp.where(i == 0, 0.0, acc[0])
    prev_c = jnp.where(i == 0, 0.0, acc[1])
    acc[0] = prev_s + bsum
    acc[1] = prev_c + bcnt

    @pl.when(i == grid - 1)
    def _():
      loss_ref[0, 0] = acc[0] / jnp.maximum(acc[1], 1.0)

  return pl.pallas_call(
      body,
      grid=(grid,),
      in_specs=[
          pl.BlockSpec((blk, _H), lambda i: (i, 0)),
          pl.BlockSpec((blk, _H), lambda i: (i, 0)),
          pl.BlockSpec((2 * _H, _H), lambda i: (0, 0)),
          pl.BlockSpec((1, _H), lambda i: (0, 0)),
          pl.BlockSpec((1, blk), lambda i: (i, 0)),
          pl.BlockSpec((1, blk), lambda i: (i, 0)),
      ],
      out_specs=[
          pl.BlockSpec((blk, _H), lambda i: (i, 0)),
          pl.BlockSpec((1, 1), lambda i: (0, 0), memory_space=pltpu.SMEM),
      ],
      out_shape=[
          jax.ShapeDtypeStruct((_N, _H), jnp.float32),
          jax.ShapeDtypeStruct((1, 1), jnp.float32),
      ],
      scratch_shapes=[pltpu.SMEM((2,), jnp.float32)],
  )(lm, g, wpad, bpad, mask2d, labels2d)


def kernel(output, edge_index, edge_type, attention_mask, labels,
           lm_W, lm_b, ln_g, ln_b, rgcn_W, rgcn_root, rgcn_bias, cls_W, cls_b):
  xflat = output.reshape(_N, _DLM)
  src = edge_index[0]
  dst = edge_index[1]

  lm = _lm_head(xflat, lm_W, lm_b, ln_g, ln_b)

  x = lm
  for l in range(_NL):
    s = _sc_segment_mean(x, src, dst, edge_type)
    x = _rgcn_dense(x, s.reshape(_R, _N, _H), rgcn_root[l], rgcn_W[l],
                    rgcn_bias[l])

  wpad = jnp.zeros((2 * _H, _H), jnp.float32).at[:, :_C].set(cls_W)
  bpad = jnp.full((1, _H), -1e30, jnp.float32).at[0, :_C].set(cls_b)
  logits_pad, loss = _cls_head(lm, x, wpad, bpad,
                               attention_mask.reshape(_N // 512, 512),
                               labels.reshape(_N // 512, 512))
  logits = logits_pad[:, :_C].reshape(_B, _SEQ, _C)
  return loss[0, 0], logits


# speedup vs baseline: 19.9486x; 19.9486x over previous
import jax
import jax.numpy as jnp
from jax import lax
from jax.experimental import pallas as pl
from jax.experimental.pallas import tpu as pltpu
from jax.experimental.pallas import tpu_sc as plsc

_B = 8
_SEQ = 2048
_DLM = 1024
_H = 128
_R = 3
_NL = 2
_C = 8
_N = _B * _SEQ
_E = 524288
_NT = 16
_W32 = 32
_ROWS = _R * _N
_EPT = _E // _NT
_BLK = 1024
_NB = _EPT // _BLK
_CH = 128
_CPB = _BLK // _CH
_RPT = _ROWS // _NT


def _memset_rows(ref, rows, value):
  def bd(i, _):
    ref[i, pl.ds(0, 16)] = jnp.full((16,), value, jnp.float32)
    ref[i, pl.ds(16, 16)] = jnp.full((16,), value, jnp.float32)
    return 0
  lax.fori_loop(0, rows, bd, 0)


def _sc_inv_counts(dst, typ):
  mesh = plsc.VectorSubcoreMesh(core_axis_name="c", subcore_axis_name="s")

  def body(dst_hbm, typ_hbm, inv_hbm, cnt, stage_d, stage_t, loc2d,
           ones, zbuf, ibuf, stgsem, csem, zsem):
    c = lax.axis_index("c")
    w = lax.axis_index("s")
    e0 = w * _EPT

    _memset_rows(ones, _CH, 1.0)
    _memset_rows(zbuf, _CH, 0.0)

    def z_start(k, _):
      pltpu.async_copy(zbuf, cnt.at[pl.ds(w * _RPT + k * _CH, _CH)], zsem)
      return 0
    lax.fori_loop(0, _RPT // _CH, z_start, 0)

    def z_wait(k, _):
      pltpu.make_async_copy(zbuf, cnt.at[pl.ds(w * _RPT + k * _CH, _CH)],
                            zsem).wait()
      return 0
    lax.fori_loop(0, _RPT // _CH, z_wait, 0)
    plsc.subcore_barrier()

    def start_stage(bi):
      sb = bi & 1
      eoff = e0 + bi * _BLK
      pltpu.async_copy(dst_hbm.at[pl.ds(eoff, _BLK)], stage_d.at[sb],
                       stgsem.at[sb])
      pltpu.async_copy(typ_hbm.at[pl.ds(eoff, _BLK)], stage_t.at[sb],
                       stgsem.at[sb])

    def wait_stage(bi):
      sb = bi & 1
      eoff = e0 + bi * _BLK
      pltpu.make_async_copy(dst_hbm.at[pl.ds(eoff, _BLK)], stage_d.at[sb],
                            stgsem.at[sb]).wait()
      pltpu.make_async_copy(typ_hbm.at[pl.ds(eoff, _BLK)], stage_t.at[sb],
                            stgsem.at[sb]).wait()

    start_stage(0)
    start_stage(1)

    def blk_body(bi, _):
      sb = bi & 1
      wait_stage(bi)

      def loc_body(ch, _):
        for k in range(8):
          d = stage_d[sb, pl.ds(ch * _CH + k * 16, 16)]
          t = stage_t[sb, pl.ds(ch * _CH + k * 16, 16)]
          loc2d[sb, ch, pl.ds(k * 16, 16)] = (t << 14) + d
        return 0
      lax.fori_loop(0, _CPB, loc_body, 0)

      def sc_start(j, _):
        pltpu.async_copy(ones, cnt.at[loc2d.at[sb, j]], csem, add=True)
        return 0
      lax.fori_loop(0, _CPB, sc_start, 0)

      def sc_wait(j, _):
        pltpu.make_async_copy(ones, cnt.at[loc2d.at[sb, j]], csem).wait()
        return 0
      lax.fori_loop(0, _CPB, sc_wait, 0)

      @pl.when(bi + 2 < _NB)
      def _():
        start_stage(bi + 2)
      return 0

    lax.fori_loop(0, _NB, blk_body, 0)
    plsc.subcore_barrier()

    half = _ROWS // 2

    def inv_chunk(k, _):
      rg = c * half + w * (half // _NT) + k * _CH
      pltpu.sync_copy(cnt.at[pl.ds(rg, _CH)], ibuf)

      def inv_row(i, _):
        ibuf[i, pl.ds(0, 16)] = 1.0 / jnp.maximum(ibuf[i, pl.ds(0, 16)], 1.0)
        ibuf[i, pl.ds(16, 16)] = 1.0 / jnp.maximum(ibuf[i, pl.ds(16, 16)], 1.0)
        return 0
      lax.fori_loop(0, _CH, inv_row, 0)
      pltpu.sync_copy(ibuf, inv_hbm.at[pl.ds(rg, _CH)])
      return 0

    lax.fori_loop(0, (half // _NT) // _CH, inv_chunk, 0)

  fn = pl.kernel(
      body,
      out_type=jax.ShapeDtypeStruct((_ROWS, _W32), jnp.float32),
      mesh=mesh,
      compiler_params=pltpu.CompilerParams(use_tc_tiling_on_sc=False),
      scratch_types=[
          pltpu.VMEM_SHARED((_ROWS, _W32), jnp.float32),
          pltpu.VMEM((2, _BLK), jnp.int32),
          pltpu.VMEM((2, _BLK), jnp.int32),
          pltpu.VMEM((2, _CPB, _CH), jnp.int32),
          pltpu.VMEM((_CH, _W32), jnp.float32),
          pltpu.VMEM((_CH, _W32), jnp.float32),
          pltpu.VMEM((_CH, _W32), jnp.float32),
          pltpu.SemaphoreType.DMA((2,)),
          pltpu.SemaphoreType.DMA,
          pltpu.SemaphoreType.DMA,
      ],
  )
  return fn(dst, typ)


def _sc_slab_sums(xs, src, dst, typ):
  mesh = plsc.VectorSubcoreMesh(core_axis_name="c", subcore_axis_name="s")

  def body(x0, x1, x2, x3, src_hbm, dst_hbm, typ_hbm, s_hbm,
           acc, stage_s, stage_d, stage_t, loc2d, gbuf, zbuf,
           stgsem, gsem, ssem, zsem):
    c = lax.axis_index("c")
    w = lax.axis_index("s")
    e0 = w * _EPT
    xrefs = [x0, x1, x2, x3]

    _memset_rows(zbuf, _CH, 0.0)

    def start_stage(bi):
      sb = bi & 1
      eoff = e0 + bi * _BLK
      pltpu.async_copy(src_hbm.at[pl.ds(eoff, _BLK)], stage_s.at[sb],
                       stgsem.at[sb])
      pltpu.async_copy(dst_hbm.at[pl.ds(eoff, _BLK)], stage_d.at[sb],
                       stgsem.at[sb])
      pltpu.async_copy(typ_hbm.at[pl.ds(eoff, _BLK)], stage_t.at[sb],
                       stgsem.at[sb])

    def wait_stage(bi):
      sb = bi & 1
      eoff = e0 + bi * _BLK
      pltpu.make_async_copy(src_hbm.at[pl.ds(eoff, _BLK)], stage_s.at[sb],
                            stgsem.at[sb]).wait()
      pltpu.make_async_copy(dst_hbm.at[pl.ds(eoff, _BLK)], stage_d.at[sb],
                            stgsem.at[sb]).wait()
      pltpu.make_async_copy(typ_hbm.at[pl.ds(eoff, _BLK)], stage_t.at[sb],
                            stgsem.at[sb]).wait()

    def slab_pass(xt, slab):
      def z_start(k, _):
        pltpu.async_copy(zbuf, acc.at[pl.ds(w * _RPT + k * _CH, _CH)], zsem)
        return 0
      lax.fori_loop(0, _RPT // _CH, z_start, 0)

      def z_wait(k, _):
        pltpu.make_async_copy(zbuf, acc.at[pl.ds(w * _RPT + k * _CH, _CH)],
                              zsem).wait()
        return 0
      lax.fori_loop(0, _RPT // _CH, z_wait, 0)
      plsc.subcore_barrier()

      start_stage(0)
      start_stage(1)

      def gstart(sb, j):
        pltpu.async_copy(xt.at[stage_s.at[sb, pl.ds(j * _CH, _CH)]],
                         gbuf.at[j & 3], gsem.at[j & 3])

      def blk_body(bi, _):
        sb = bi & 1
        wait_stage(bi)

        def loc_body(ch, _):
          for k in range(8):
            d = stage_d[sb, pl.ds(ch * _CH + k * 16, 16)]
            t = stage_t[sb, pl.ds(ch * _CH + k * 16, 16)]
            loc2d[sb, ch, pl.ds(k * 16, 16)] = (t << 14) + d
          return 0
        lax.fori_loop(0, _CPB, loc_body, 0)

        for j in range(4):
          gstart(sb, j)

        def chunk_body(j, _):
          pltpu.make_async_copy(xt.at[stage_s.at[sb, pl.ds(j * _CH, _CH)]],
                                gbuf.at[j & 3], gsem.at[j & 3]).wait()
          pltpu.async_copy(gbuf.at[j & 3], acc.at[loc2d.at[sb, j]],
                           ssem.at[j & 1], add=True)

          @pl.when(j >= 1)
          def _():
            pltpu.make_async_copy(gbuf.at[(j - 1) & 3],
                                  acc.at[loc2d.at[sb, j - 1]],
                                  ssem.at[(j - 1) & 1]).wait()

          @pl.when(j + 4 < _CPB)
          def _():
            gstart(sb, j + 4)
          return 0

        lax.fori_loop(0, _CPB, chunk_body, 0)
        pltpu.make_async_copy(gbuf.at[(_CPB - 1) & 3],
                              acc.at[loc2d.at[sb, _CPB - 1]],
                              ssem.at[(_CPB - 1) & 1]).wait()

        @pl.when(bi + 2 < _NB)
        def _():
          start_stage(bi + 2)
        return 0

      lax.fori_loop(0, _NB, blk_body, 0)
      plsc.subcore_barrier()
      pltpu.sync_copy(acc.at[pl.ds(w * _RPT, _RPT)],
                      s_hbm.at[slab, pl.ds(w * _RPT, _RPT)])
      plsc.subcore_barrier()

    for slab in range(4):
      @pl.when(c == slab // 2)
      def _(slab=slab):
        slab_pass(xrefs[slab], slab)

  fn = pl.kernel(
      body,
      out_type=jax.ShapeDtypeStruct((4, _ROWS, _W32), jnp.float32),
      mesh=mesh,
      compiler_params=pltpu.CompilerParams(use_tc_tiling_on_sc=False),
      scratch_types=[
          pltpu.VMEM_SHARED((_ROWS, _W32), jnp.float32),
          pltpu.VMEM((2, _BLK), jnp.int32),
          pltpu.VMEM((2, _BLK), jnp.int32),
          pltpu.VMEM((2, _BLK), jnp.int32),
          pltpu.VMEM((2, _CPB, _CH), jnp.int32),
          pltpu.VMEM((4, _CH, _W32), jnp.float32),
          pltpu.VMEM((_CH, _W32), jnp.float32),
          pltpu.SemaphoreType.DMA((2,)),
          pltpu.SemaphoreType.DMA((4,)),
          pltpu.SemaphoreType.DMA((2,)),
          pltpu.SemaphoreType.DMA,
      ],
  )
  return fn(xs[0], xs[1], xs[2], xs[3], src, dst, typ)


def _lm_head(xflat, lm_W, lm_b, ln_g, ln_b):
  blk = 256
  grid = _N // blk

  def body(x_ref, w_ref, b_ref, g_ref, lb_ref, o_ref, s0, s1, s2, s3):
    h = jnp.dot(x_ref[...], w_ref[...], preferred_element_type=jnp.float32)
    h = jnp.maximum(h + b_ref[...], 0.0)
    mu = jnp.mean(h, axis=-1, keepdims=True)
    var = jnp.mean((h - mu) ** 2, axis=-1, keepdims=True)
    o = (h - mu) * lax.rsqrt(var + 1e-5) * g_ref[...] + lb_ref[...]
    o_ref[...] = o
    for s, ref in enumerate((s0, s1, s2, s3)):
      ref[...] = o[:, s * _W32:(s + 1) * _W32]

  xs_spec = pl.BlockSpec((blk, _W32), lambda i: (i, 0))
  xs_shape = jax.ShapeDtypeStruct((_N, _W32), jnp.float32)
  return pl.pallas_call(
      body,
      grid=(grid,),
      in_specs=[
          pl.BlockSpec((blk, _DLM), lambda i: (i, 0)),
          pl.BlockSpec((_DLM, _H), lambda i: (0, 0)),
          pl.BlockSpec((1, _H), lambda i: (0, 0)),
          pl.BlockSpec((1, _H), lambda i: (0, 0)),
          pl.BlockSpec((1, _H), lambda i: (0, 0)),
      ],
      out_specs=[pl.BlockSpec((blk, _H), lambda i: (i, 0))] + [xs_spec] * 4,
      out_shape=[jax.ShapeDtypeStruct((_N, _H), jnp.float32)] + [xs_shape] * 4,
  )(xflat, lm_W, lm_b.reshape(1, _H), ln_g.reshape(1, _H), ln_b.reshape(1, _H))


def _rgcn_dense(x, s4, inv3, root, wr, bias, emit_slabs):
  blk = 256
  grid = _N // blk

  def body(x_ref, s_ref, i_ref, r_ref, w_ref, b_ref, *outs):
    o = jnp.dot(x_ref[...], r_ref[...], preferred_element_type=jnp.float32)
    o = o + b_ref[...]
    for r in range(_R):
      iv = i_ref[r]
      sfull = jnp.concatenate([s_ref[s, r] * iv for s in range(4)], axis=-1)
      o = o + jnp.dot(sfull, w_ref[r], preferred_element_type=jnp.float32)
    o = jnp.maximum(o, 0.0)
    outs[0][...] = o
    if emit_slabs:
      for s in range(4):
        outs[1 + s][...] = o[:, s * _W32:(s + 1) * _W32]

  out_specs = [pl.BlockSpec((blk, _H), lambda i: (i, 0))]
  out_shape = [jax.ShapeDtypeStruct((_N, _H), jnp.float32)]
  if emit_slabs:
    out_specs += [pl.BlockSpec((blk, _W32), lambda i: (i, 0))] * 4
    out_shape += [jax.ShapeDtypeStruct((_N, _W32), jnp.float32)] * 4
  return pl.pallas_call(
      body,
      grid=(grid,),
      in_specs=[
          pl.BlockSpec((blk, _H), lambda i: (i, 0)),
          pl.BlockSpec((4, _R, blk, _W32), lambda i: (0, 0, i, 0)),
          pl.BlockSpec((_R, blk, _W32), lambda i: (0, i, 0)),
          pl.BlockSpec((_H, _H), lambda i: (0, 0)),
          pl.BlockSpec((_R, _H, _H), lambda i: (0, 0, 0)),
          pl.BlockSpec((1, _H), lambda i: (0, 0)),
      ],
      out_specs=out_specs,
      out_shape=out_shape,
  )(x, s4, inv3, root, wr, bias.reshape(1, _H))


def _cls_head(lm, g, wpad, bpad, mask3d, labels3d):
  blk = 512
  grid = _N // blk

  def body(lm_ref, g_ref, w_ref, b_ref, m_ref, l_ref, lo_ref, loss_ref, acc):
    i = pl.program_id(0)
    logits = (jnp.dot(lm_ref[...], w_ref[0:_H], preferred_element_type=jnp.float32)
              + jnp.dot(g_ref[...], w_ref[_H:2 * _H], preferred_element_type=jnp.float32)
              + b_ref[...])
    lo_ref[...] = logits
    mx = jnp.max(logits, axis=-1, keepdims=True)
    lse = jnp.log(jnp.sum(jnp.exp(logits - mx), axis=-1, keepdims=True)) + mx
    lab = l_ref[0, 0, :]
    cols = lax.broadcasted_iota(jnp.int32, (blk, _H), 1)
    pick = jnp.sum(jnp.where(cols == lab[:, None], logits, 0.0), axis=-1,
                   keepdims=True)
    active = (m_ref[0, 0, :] == 1).astype(jnp.float32)[:, None]
    bsum = jnp.sum((lse - pick) * active)
    bcnt = jnp.sum(active)
    prev_s = jnp.where(i == 0, 0.0, acc[0])
    prev_c = jnp.where(i == 0, 0.0, acc[1])
    acc[0] = prev_s + bsum
    acc[1] = prev_c + bcnt

    @pl.when(i == grid - 1)
    def _():
      loss_ref[0, 0] = acc[0] / jnp.maximum(acc[1], 1.0)

  return pl.pallas_call(
      body,
      grid=(grid,),
      in_specs=[
          pl.BlockSpec((blk, _H), lambda i: (i, 0)),
          pl.BlockSpec((blk, _H), lambda i: (i, 0)),
          pl.BlockSpec((2 * _H, _H), lambda i: (0, 0)),
          pl.BlockSpec((1, _H), lambda i: (0, 0)),
          pl.BlockSpec((1, 1, blk), lambda i: (i, 0, 0)),
          pl.BlockSpec((1, 1, blk), lambda i: (i, 0, 0)),
      ],
      out_specs=[
          pl.BlockSpec((blk, _H), lambda i: (i, 0)),
          pl.BlockSpec((1, 1), lambda i: (0, 0), memory_space=pltpu.SMEM),
      ],
      out_shape=[
          jax.ShapeDtypeStruct((_N, _H), jnp.float32),
          jax.ShapeDtypeStruct((1, 1), jnp.float32),
      ],
      scratch_shapes=[pltpu.SMEM((2,), jnp.float32)],
  )(lm, g, wpad, bpad, mask3d, labels3d)


def kernel(output, edge_index, edge_type, attention_mask, labels,
           lm_W, lm_b, ln_g, ln_b, rgcn_W, rgcn_root, rgcn_bias, cls_W, cls_b):
  xflat = output.reshape(_N, _DLM)
  src = edge_index[0]
  dst = edge_index[1]

  lm, a0, a1, a2, a3 = _lm_head(xflat, lm_W, lm_b, ln_g, ln_b)
  inv = _sc_inv_counts(dst, edge_type)
  inv3 = inv.reshape(_R, _N, _W32)

  s0 = _sc_slab_sums((a0, a1, a2, a3), src, dst, edge_type)
  x1, b0, b1, b2, b3 = _rgcn_dense(lm, s0.reshape(4, _R, _N, _W32), inv3,
                                   rgcn_root[0], rgcn_W[0], rgcn_bias[0],
                                   emit_slabs=True)
  s1 = _sc_slab_sums((b0, b1, b2, b3), src, dst, edge_type)
  (x2,) = _rgcn_dense(x1, s1.reshape(4, _R, _N, _W32), inv3,
                      rgcn_root[1], rgcn_W[1], rgcn_bias[1], emit_slabs=False)

  wpad = jnp.zeros((2 * _H, _H), jnp.float32).at[:, :_C].set(cls_W)
  bpad = jnp.full((1, _H), -1e30, jnp.float32).at[0, :_C].set(cls_b)
  logits_pad, loss = _cls_head(lm, x2, wpad, bpad,
                               attention_mask.reshape(_N // 512, 1, 512),
                               labels.reshape(_N // 512, 1, 512))
  logits = logits_pad[:, :_C].reshape(_B, _SEQ, _C)
  return loss[0, 0], logits


# speedup vs baseline: 20.2908x; 1.0172x over previous
import jax
import jax.numpy as jnp
from jax import lax
from jax.experimental import pallas as pl
from jax.experimental.pallas import tpu as pltpu
from jax.experimental.pallas import tpu_sc as plsc

_B = 8
_SEQ = 2048
_DLM = 1024
_H = 128
_R = 3
_NL = 2
_C = 8
_N = _B * _SEQ
_E = 524288
_NT = 16
_W32 = 32
_ROWS = _R * _N
_EPT = _E // _NT
_BLK = 2048
_NB = _EPT // _BLK
_CH = 128
_CPB = _BLK // _CH
_RPT = _ROWS // _NT


def _memset_rows(ref, rows, value):
  def bd(i, _):
    ref[i, pl.ds(0, 16)] = jnp.full((16,), value, jnp.float32)
    ref[i, pl.ds(16, 16)] = jnp.full((16,), value, jnp.float32)
    return 0
  lax.fori_loop(0, rows, bd, 0)


def _sc_inv_counts(dst, typ):
  mesh = plsc.VectorSubcoreMesh(core_axis_name="c", subcore_axis_name="s")

  def body(dst_hbm, typ_hbm, zeros_hbm, inv_hbm, cnt, stage_d, stage_t, loc2d,
           ones, ibuf, stgsem, csem, zsem):
    c = lax.axis_index("c")
    w = lax.axis_index("s")
    e0 = w * _EPT

    _memset_rows(ones, _CH, 1.0)

    pltpu.async_copy(zeros_hbm, cnt.at[pl.ds(w * _RPT, _RPT)], zsem).wait()
    plsc.subcore_barrier()

    def start_stage(bi):
      sb = bi & 1
      eoff = e0 + bi * _BLK
      pltpu.async_copy(dst_hbm.at[pl.ds(eoff, _BLK)], stage_d.at[sb],
                       stgsem.at[sb])
      pltpu.async_copy(typ_hbm.at[pl.ds(eoff, _BLK)], stage_t.at[sb],
                       stgsem.at[sb])

    def wait_stage(bi):
      sb = bi & 1
      eoff = e0 + bi * _BLK
      pltpu.make_async_copy(dst_hbm.at[pl.ds(eoff, _BLK)], stage_d.at[sb],
                            stgsem.at[sb]).wait()
      pltpu.make_async_copy(typ_hbm.at[pl.ds(eoff, _BLK)], stage_t.at[sb],
                            stgsem.at[sb]).wait()

    start_stage(0)
    start_stage(1)

    def blk_body(bi, _):
      sb = bi & 1
      wait_stage(bi)

      def loc_body(ch, _):
        for k in range(8):
          d = stage_d[sb, pl.ds(ch * _CH + k * 16, 16)]
          t = stage_t[sb, pl.ds(ch * _CH + k * 16, 16)]
          loc2d[ch, pl.ds(k * 16, 16)] = (t << 14) + d
        return 0
      lax.fori_loop(0, _CPB, loc_body, 0)

      def sc_start(j, _):
        pltpu.async_copy(ones, cnt.at[loc2d.at[j]], csem, add=True)
        return 0
      lax.fori_loop(0, _CPB, sc_start, 0)

      def sc_wait(j, _):
        pltpu.make_async_copy(ones, cnt.at[loc2d.at[j]], csem).wait()
        return 0
      lax.fori_loop(0, _CPB, sc_wait, 0)

      @pl.when(bi + 2 < _NB)
      def _():
        start_stage(bi + 2)
      return 0

    lax.fori_loop(0, _NB, blk_body, 0)
    plsc.subcore_barrier()

    half = _ROWS // 2

    def inv_chunk(k, _):
      rg = c * half + w * (half // _NT) + k * _CH
      pltpu.sync_copy(cnt.at[pl.ds(rg, _CH)], ibuf)

      def inv_row(i, _):
        ibuf[i, pl.ds(0, 16)] = 1.0 / jnp.maximum(ibuf[i, pl.ds(0, 16)], 1.0)
        ibuf[i, pl.ds(16, 16)] = 1.0 / jnp.maximum(ibuf[i, pl.ds(16, 16)], 1.0)
        return 0
      lax.fori_loop(0, _CH, inv_row, 0)
      pltpu.sync_copy(ibuf, inv_hbm.at[pl.ds(rg, _CH)])
      return 0

    lax.fori_loop(0, (half // _NT) // _CH, inv_chunk, 0)

  fn = pl.kernel(
      body,
      out_type=jax.ShapeDtypeStruct((_ROWS, _W32), jnp.float32),
      mesh=mesh,
      compiler_params=pltpu.CompilerParams(use_tc_tiling_on_sc=False),
      scratch_types=[
          pltpu.VMEM_SHARED((_ROWS, _W32), jnp.float32),
          pltpu.VMEM((2, _BLK), jnp.int32),
          pltpu.VMEM((2, _BLK), jnp.int32),
          pltpu.VMEM((_CPB, _CH), jnp.int32),
          pltpu.VMEM((_CH, _W32), jnp.float32),
          pltpu.VMEM((_CH, _W32), jnp.float32),
          pltpu.SemaphoreType.DMA((2,)),
          pltpu.SemaphoreType.DMA,
          pltpu.SemaphoreType.DMA,
      ],
  )
  return fn(dst, typ, jnp.zeros((_RPT, _W32), jnp.float32))


def _sc_slab_sums(xs, src, dst, typ):
  mesh = plsc.VectorSubcoreMesh(core_axis_name="c", subcore_axis_name="s")

  def body(x0, x1, x2, x3, src_hbm, dst_hbm, typ_hbm, zeros_hbm, s_hbm,
           acc, stage_s, stage_d, stage_t, loc2d, gbuf,
           stgsem, gsem, ssem, zsem):
    c = lax.axis_index("c")
    w = lax.axis_index("s")
    e0 = w * _EPT
    xrefs = [x0, x1, x2, x3]

    def start_stage(bi):
      sb = bi & 1
      eoff = e0 + bi * _BLK
      pltpu.async_copy(src_hbm.at[pl.ds(eoff, _BLK)], stage_s.at[sb],
                       stgsem.at[sb])
      pltpu.async_copy(dst_hbm.at[pl.ds(eoff, _BLK)], stage_d.at[sb],
                       stgsem.at[sb])
      pltpu.async_copy(typ_hbm.at[pl.ds(eoff, _BLK)], stage_t.at[sb],
                       stgsem.at[sb])

    def wait_stage(bi):
      sb = bi & 1
      eoff = e0 + bi * _BLK
      pltpu.make_async_copy(src_hbm.at[pl.ds(eoff, _BLK)], stage_s.at[sb],
                            stgsem.at[sb]).wait()
      pltpu.make_async_copy(dst_hbm.at[pl.ds(eoff, _BLK)], stage_d.at[sb],
                            stgsem.at[sb]).wait()
      pltpu.make_async_copy(typ_hbm.at[pl.ds(eoff, _BLK)], stage_t.at[sb],
                            stgsem.at[sb]).wait()

    def slab_pass(xt, slab):
      pltpu.async_copy(zeros_hbm, acc.at[pl.ds(w * _RPT, _RPT)], zsem).wait()
      plsc.subcore_barrier()

      start_stage(0)
      start_stage(1)

      def gstart(sb, j):
        pltpu.async_copy(xt.at[stage_s.at[sb, pl.ds(j * _CH, _CH)]],
                         gbuf.at[j & 3], gsem.at[j & 3])

      def blk_body(bi, _):
        sb = bi & 1
        wait_stage(bi)

        def loc_body(ch, _):
          for k in range(8):
            d = stage_d[sb, pl.ds(ch * _CH + k * 16, 16)]
            t = stage_t[sb, pl.ds(ch * _CH + k * 16, 16)]
            loc2d[ch, pl.ds(k * 16, 16)] = (t << 14) + d
          return 0
        lax.fori_loop(0, _CPB, loc_body, 0)

        for j in range(4):
          gstart(sb, j)

        def chunk_body(j, _):
          pltpu.make_async_copy(xt.at[stage_s.at[sb, pl.ds(j * _CH, _CH)]],
                                gbuf.at[j & 3], gsem.at[j & 3]).wait()
          pltpu.async_copy(gbuf.at[j & 3], acc.at[loc2d.at[j]],
                           ssem.at[j & 1], add=True)

          @pl.when(j >= 1)
          def _():
            pltpu.make_async_copy(gbuf.at[(j - 1) & 3],
                                  acc.at[loc2d.at[j - 1]],
                                  ssem.at[(j - 1) & 1]).wait()

          @pl.when(j + 4 < _CPB)
          def _():
            gstart(sb, j + 4)
          return 0

        lax.fori_loop(0, _CPB, chunk_body, 0)
        pltpu.make_async_copy(gbuf.at[(_CPB - 1) & 3],
                              acc.at[loc2d.at[_CPB - 1]],
                              ssem.at[(_CPB - 1) & 1]).wait()

        @pl.when(bi + 2 < _NB)
        def _():
          start_stage(bi + 2)
        return 0

      lax.fori_loop(0, _NB, blk_body, 0)
      plsc.subcore_barrier()
      pltpu.sync_copy(acc.at[pl.ds(w * _RPT, _RPT)],
                      s_hbm.at[slab, pl.ds(w * _RPT, _RPT)])
      plsc.subcore_barrier()

    for slab in range(4):
      @pl.when(c == slab // 2)
      def _(slab=slab):
        slab_pass(xrefs[slab], slab)

  fn = pl.kernel(
      body,
      out_type=jax.ShapeDtypeStruct((4, _ROWS, _W32), jnp.float32),
      mesh=mesh,
      compiler_params=pltpu.CompilerParams(use_tc_tiling_on_sc=False),
      scratch_types=[
          pltpu.VMEM_SHARED((_ROWS, _W32), jnp.float32),
          pltpu.VMEM((2, _BLK), jnp.int32),
          pltpu.VMEM((2, _BLK), jnp.int32),
          pltpu.VMEM((2, _BLK), jnp.int32),
          pltpu.VMEM((_CPB, _CH), jnp.int32),
          pltpu.VMEM((4, _CH, _W32), jnp.float32),
          pltpu.SemaphoreType.DMA((2,)),
          pltpu.SemaphoreType.DMA((4,)),
          pltpu.SemaphoreType.DMA((2,)),
          pltpu.SemaphoreType.DMA,
      ],
  )
  return fn(xs[0], xs[1], xs[2], xs[3], src, dst, typ,
            jnp.zeros((_RPT, _W32), jnp.float32))


def _lm_head(xflat, lm_W, lm_b, ln_g, ln_b):
  blk = 256
  grid = _N // blk

  def body(x_ref, w_ref, b_ref, g_ref, lb_ref, o_ref, s0, s1, s2, s3):
    h = jnp.dot(x_ref[...], w_ref[...], preferred_element_type=jnp.float32)
    h = jnp.maximum(h + b_ref[...], 0.0)
    mu = jnp.mean(h, axis=-1, keepdims=True)
    var = jnp.mean((h - mu) ** 2, axis=-1, keepdims=True)
    o = (h - mu) * lax.rsqrt(var + 1e-5) * g_ref[...] + lb_ref[...]
    o_ref[...] = o
    for s, ref in enumerate((s0, s1, s2, s3)):
      ref[...] = o[:, s * _W32:(s + 1) * _W32]

  xs_spec = pl.BlockSpec((blk, _W32), lambda i: (i, 0))
  xs_shape = jax.ShapeDtypeStruct((_N, _W32), jnp.float32)
  return pl.pallas_call(
      body,
      grid=(grid,),
      in_specs=[
          pl.BlockSpec((blk, _DLM), lambda i: (i, 0)),
          pl.BlockSpec((_DLM, _H), lambda i: (0, 0)),
          pl.BlockSpec((1, _H), lambda i: (0, 0)),
          pl.BlockSpec((1, _H), lambda i: (0, 0)),
          pl.BlockSpec((1, _H), lambda i: (0, 0)),
      ],
      out_specs=[pl.BlockSpec((blk, _H), lambda i: (i, 0))] + [xs_spec] * 4,
      out_shape=[jax.ShapeDtypeStruct((_N, _H), jnp.float32)] + [xs_shape] * 4,
  )(xflat, lm_W, lm_b.reshape(1, _H), ln_g.reshape(1, _H), ln_b.reshape(1, _H))


def _rgcn_dense(x, s4, inv3, root, wr, bias, emit_slabs):
  blk = 256
  grid = _N // blk

  def body(x_ref, s_ref, i_ref, r_ref, w_ref, b_ref, *outs):
    o = jnp.dot(x_ref[...], r_ref[...], preferred_element_type=jnp.float32)
    o = o + b_ref[...]
    for r in range(_R):
      iv1 = i_ref[r][:, 0:1]
      acc_r = jnp.dot(s_ref[0, r], w_ref[r, 0 * _W32:1 * _W32],
                      preferred_element_type=jnp.float32)
      for s in range(1, 4):
        acc_r = acc_r + jnp.dot(s_ref[s, r], w_ref[r, s * _W32:(s + 1) * _W32],
                                preferred_element_type=jnp.float32)
      o = o + iv1 * acc_r
    o = jnp.maximum(o, 0.0)
    outs[0][...] = o
    if emit_slabs:
      for s in range(4):
        outs[1 + s][...] = o[:, s * _W32:(s + 1) * _W32]

  out_specs = [pl.BlockSpec((blk, _H), lambda i: (i, 0))]
  out_shape = [jax.ShapeDtypeStruct((_N, _H), jnp.float32)]
  if emit_slabs:
    out_specs += [pl.BlockSpec((blk, _W32), lambda i: (i, 0))] * 4
    out_shape += [jax.ShapeDtypeStruct((_N, _W32), jnp.float32)] * 4
  return pl.pallas_call(
      body,
      grid=(grid,),
      in_specs=[
          pl.BlockSpec((blk, _H), lambda i: (i, 0)),
          pl.BlockSpec((4, _R, blk, _W32), lambda i: (0, 0, i, 0)),
          pl.BlockSpec((_R, blk, _W32), lambda i: (0, i, 0)),
          pl.BlockSpec((_H, _H), lambda i: (0, 0)),
          pl.BlockSpec((_R, _H, _H), lambda i: (0, 0, 0)),
          pl.BlockSpec((1, _H), lambda i: (0, 0)),
      ],
      out_specs=out_specs,
      out_shape=out_shape,
  )(x, s4, inv3, root, wr, bias.reshape(1, _H))


def _cls_head(lm, g, wpad, bpad, mask3d, labels3d):
  blk = 512
  grid = _N // blk

  def body(lm_ref, g_ref, w_ref, b_ref, m_ref, l_ref, lo_ref, loss_ref, acc):
    i = pl.program_id(0)
    logits = (jnp.dot(lm_ref[...], w_ref[0:_H], preferred_element_type=jnp.float32)
              + jnp.dot(g_ref[...], w_ref[_H:2 * _H], preferred_element_type=jnp.float32)
              + b_ref[...])
    lo_ref[...] = logits
    mx = jnp.max(logits, axis=-1, keepdims=True)
    lse = jnp.log(jnp.sum(jnp.exp(logits - mx), axis=-1, keepdims=True)) + mx
    lab = l_ref[0, 0, :]
    cols = lax.broadcasted_iota(jnp.int32, (blk, _H), 1)
    pick = jnp.sum(jnp.where(cols == lab[:, None], logits, 0.0), axis=-1,
                   keepdims=True)
    active = (m_ref[0, 0, :] == 1).astype(jnp.float32)[:, None]
    bsum = jnp.sum((lse - pick) * active)
    bcnt = jnp.sum(active)
    prev_s = jnp.where(i == 0, 0.0, acc[0])
    prev_c = jnp.where(i == 0, 0.0, acc[1])
    acc[0] = prev_s + bsum
    acc[1] = prev_c + bcnt

    @pl.when(i == grid - 1)
    def _():
      loss_ref[0, 0] = acc[0] / jnp.maximum(acc[1], 1.0)

  return pl.pallas_call(
      body,
      grid=(grid,),
      in_specs=[
          pl.BlockSpec((blk, _H), lambda i: (i, 0)),
          pl.BlockSpec((blk, _H), lambda i: (i, 0)),
          pl.BlockSpec((2 * _H, _H), lambda i: (0, 0)),
          pl.BlockSpec((1, _H), lambda i: (0, 0)),
          pl.BlockSpec((1, 1, blk), lambda i: (i, 0, 0)),
          pl.BlockSpec((1, 1, blk), lambda i: (i, 0, 0)),
      ],
      out_specs=[
          pl.BlockSpec((blk, _H), lambda i: (i, 0)),
          pl.BlockSpec((1, 1), lambda i: (0, 0), memory_space=pltpu.SMEM),
      ],
      out_shape=[
          jax.ShapeDtypeStruct((_N, _H), jnp.float32),
          jax.ShapeDtypeStruct((1, 1), jnp.float32),
      ],
      scratch_shapes=[pltpu.SMEM((2,), jnp.float32)],
  )(lm, g, wpad, bpad, mask3d, labels3d)


def kernel(output, edge_index, edge_type, attention_mask, labels,
           lm_W, lm_b, ln_g, ln_b, rgcn_W, rgcn_root, rgcn_bias, cls_W, cls_b):
  xflat = output.reshape(_N, _DLM)
  src = edge_index[0]
  dst = edge_index[1]

  lm, a0, a1, a2, a3 = _lm_head(xflat, lm_W, lm_b, ln_g, ln_b)
  inv = _sc_inv_counts(dst, edge_type)
  inv3 = inv.reshape(_R, _N, _W32)

  s0 = _sc_slab_sums((a0, a1, a2, a3), src, dst, edge_type)
  x1, b0, b1, b2, b3 = _rgcn_dense(lm, s0.reshape(4, _R, _N, _W32), inv3,
                                   rgcn_root[0], rgcn_W[0], rgcn_bias[0],
                                   emit_slabs=True)
  s1 = _sc_slab_sums((b0, b1, b2, b3), src, dst, edge_type)
  (x2,) = _rgcn_dense(x1, s1.reshape(4, _R, _N, _W32), inv3,
                      rgcn_root[1], rgcn_W[1], rgcn_bias[1], emit_slabs=False)

  wpad = jnp.zeros((2 * _H, _H), jnp.float32).at[:, :_C].set(cls_W)
  bpad = jnp.full((1, _H), -1e30, jnp.float32).at[0, :_C].set(cls_b)
  logits_pad, loss = _cls_head(lm, x2, wpad, bpad,
                               attention_mask.reshape(_N // 512, 1, 512),
                               labels.reshape(_N // 512, 1, 512))
  logits = logits_pad[:, :_C].reshape(_B, _SEQ, _C)
  return loss[0, 0], logits


# speedup vs baseline: 23.9620x; 1.1809x over previous
import jax
import jax.numpy as jnp
from jax import lax
from jax.experimental import pallas as pl
from jax.experimental.pallas import tpu as pltpu
from jax.experimental.pallas import tpu_sc as plsc

_B = 8
_SEQ = 2048
_DLM = 1024
_H = 128
_R = 3
_NL = 2
_C = 8
_N = _B * _SEQ
_E = 524288
_NT = 16
_W32 = 32
_ROWS = _R * _N
_EPT = _E // _NT
_BLK = 2048
_NB = _EPT // _BLK
_CH = 128
_CPB = _BLK // _CH
_RPT = _ROWS // _NT


def _memset_rows(ref, rows, value):
  def bd(i, _):
    ref[i, pl.ds(0, 16)] = jnp.full((16,), value, jnp.float32)
    ref[i, pl.ds(16, 16)] = jnp.full((16,), value, jnp.float32)
    return 0
  lax.fori_loop(0, rows, bd, 0)


def _sc_inv_counts(dst, typ):
  mesh = plsc.VectorSubcoreMesh(core_axis_name="c", subcore_axis_name="s")

  def body(dst_hbm, typ_hbm, zeros_hbm, inv_hbm, cnt, stage_d, stage_t, loc2d,
           ones, ibuf, stgsem, csem, zsem):
    c = lax.axis_index("c")
    w = lax.axis_index("s")
    e0 = w * _EPT

    _memset_rows(ones, _CH, 1.0)

    pltpu.async_copy(zeros_hbm, cnt.at[pl.ds(w * _RPT, _RPT)], zsem).wait()
    plsc.subcore_barrier()

    def start_stage(bi):
      sb = bi & 1
      eoff = e0 + bi * _BLK
      pltpu.async_copy(dst_hbm.at[pl.ds(eoff, _BLK)], stage_d.at[sb],
                       stgsem.at[sb])
      pltpu.async_copy(typ_hbm.at[pl.ds(eoff, _BLK)], stage_t.at[sb],
                       stgsem.at[sb])

    def wait_stage(bi):
      sb = bi & 1
      eoff = e0 + bi * _BLK
      pltpu.make_async_copy(dst_hbm.at[pl.ds(eoff, _BLK)], stage_d.at[sb],
                            stgsem.at[sb]).wait()
      pltpu.make_async_copy(typ_hbm.at[pl.ds(eoff, _BLK)], stage_t.at[sb],
                            stgsem.at[sb]).wait()

    start_stage(0)
    start_stage(1)

    def blk_body(bi, _):
      sb = bi & 1
      wait_stage(bi)

      def loc_body(ch, _):
        for k in range(8):
          d = stage_d[sb, pl.ds(ch * _CH + k * 16, 16)]
          t = stage_t[sb, pl.ds(ch * _CH + k * 16, 16)]
          loc2d[ch, pl.ds(k * 16, 16)] = (t << 14) + d
        return 0
      lax.fori_loop(0, _CPB, loc_body, 0)

      def sc_start(j, _):
        pltpu.async_copy(ones, cnt.at[loc2d.at[j]], csem, add=True)
        return 0
      lax.fori_loop(0, _CPB, sc_start, 0)

      def sc_wait(j, _):
        pltpu.make_async_copy(ones, cnt.at[loc2d.at[j]], csem).wait()
        return 0
      lax.fori_loop(0, _CPB, sc_wait, 0)

      @pl.when(bi + 2 < _NB)
      def _():
        start_stage(bi + 2)
      return 0

    lax.fori_loop(0, _NB, blk_body, 0)
    plsc.subcore_barrier()

    half = _ROWS // 2

    def inv_chunk(k, _):
      rg = c * half + w * (half // _NT) + k * _CH
      pltpu.sync_copy(cnt.at[pl.ds(rg, _CH)], ibuf)

      def inv_row(i, _):
        ibuf[i, pl.ds(0, 16)] = 1.0 / jnp.maximum(ibuf[i, pl.ds(0, 16)], 1.0)
        ibuf[i, pl.ds(16, 16)] = 1.0 / jnp.maximum(ibuf[i, pl.ds(16, 16)], 1.0)
        return 0
      lax.fori_loop(0, _CH, inv_row, 0)
      pltpu.sync_copy(ibuf, inv_hbm.at[pl.ds(rg, _CH)])
      return 0

    lax.fori_loop(0, (half // _NT) // _CH, inv_chunk, 0)

  fn = pl.kernel(
      body,
      out_type=jax.ShapeDtypeStruct((_ROWS, _W32), jnp.float32),
      mesh=mesh,
      compiler_params=pltpu.CompilerParams(use_tc_tiling_on_sc=False),
      scratch_types=[
          pltpu.VMEM_SHARED((_ROWS, _W32), jnp.float32),
          pltpu.VMEM((2, _BLK), jnp.int32),
          pltpu.VMEM((2, _BLK), jnp.int32),
          pltpu.VMEM((_CPB, _CH), jnp.int32),
          pltpu.VMEM((_CH, _W32), jnp.float32),
          pltpu.VMEM((_CH, _W32), jnp.float32),
          pltpu.SemaphoreType.DMA((2,)),
          pltpu.SemaphoreType.DMA,
          pltpu.SemaphoreType.DMA,
      ],
  )
  return fn(dst, typ, jnp.zeros((_RPT, _W32), jnp.float32))


def _sc_slab_sums(xs, src, dst, typ):
  mesh = plsc.VectorSubcoreMesh(core_axis_name="c", subcore_axis_name="s")

  def body(x0, x1, x2, x3, src_hbm, dst_hbm, typ_hbm, zeros_hbm, s_hbm,
           acc, stage_s, stage_d, stage_t, loc2d, gbuf,
           stgsem, gsem, ssem, zsem):
    c = lax.axis_index("c")
    w = lax.axis_index("s")
    e0 = w * _EPT
    xrefs = [x0, x1, x2, x3]

    def start_stage(bi):
      sb = bi & 1
      eoff = e0 + bi * _BLK
      pltpu.async_copy(src_hbm.at[pl.ds(eoff, _BLK)], stage_s.at[sb],
                       stgsem.at[sb])
      pltpu.async_copy(dst_hbm.at[pl.ds(eoff, _BLK)], stage_d.at[sb],
                       stgsem.at[sb])
      pltpu.async_copy(typ_hbm.at[pl.ds(eoff, _BLK)], stage_t.at[sb],
                       stgsem.at[sb])

    def wait_stage(bi):
      sb = bi & 1
      eoff = e0 + bi * _BLK
      pltpu.make_async_copy(src_hbm.at[pl.ds(eoff, _BLK)], stage_s.at[sb],
                            stgsem.at[sb]).wait()
      pltpu.make_async_copy(dst_hbm.at[pl.ds(eoff, _BLK)], stage_d.at[sb],
                            stgsem.at[sb]).wait()
      pltpu.make_async_copy(typ_hbm.at[pl.ds(eoff, _BLK)], stage_t.at[sb],
                            stgsem.at[sb]).wait()

    def slab_pass(xt, slab):
      pltpu.async_copy(zeros_hbm, acc.at[pl.ds(w * _RPT, _RPT)], zsem).wait()
      plsc.subcore_barrier()

      start_stage(0)
      start_stage(1)

      def gstart(sb, j):
        pltpu.async_copy(xt.at[stage_s.at[sb, pl.ds(j * _CH, _CH)]],
                         gbuf.at[j & 3], gsem.at[j & 3])

      def blk_body(bi, _):
        sb = bi & 1
        wait_stage(bi)

        def loc_body(ch, _):
          for k in range(8):
            d = stage_d[sb, pl.ds(ch * _CH + k * 16, 16)]
            t = stage_t[sb, pl.ds(ch * _CH + k * 16, 16)]
            loc2d[ch, pl.ds(k * 16, 16)] = (t << 14) + d
          return 0
        lax.fori_loop(0, _CPB, loc_body, 0)

        for j in range(4):
          gstart(sb, j)

        def chunk_body(j, _):
          pltpu.make_async_copy(xt.at[stage_s.at[sb, pl.ds(j * _CH, _CH)]],
                                gbuf.at[j & 3], gsem.at[j & 3]).wait()
          pltpu.async_copy(gbuf.at[j & 3], acc.at[loc2d.at[j]],
                           ssem.at[j & 1], add=True)

          @pl.when(j >= 1)
          def _():
            pltpu.make_async_copy(gbuf.at[(j - 1) & 3],
                                  acc.at[loc2d.at[j - 1]],
                                  ssem.at[(j - 1) & 1]).wait()

          @pl.when(j + 4 < _CPB)
          def _():
            gstart(sb, j + 4)
          return 0

        lax.fori_loop(0, _CPB, chunk_body, 0)
        pltpu.make_async_copy(gbuf.at[(_CPB - 1) & 3],
                              acc.at[loc2d.at[_CPB - 1]],
                              ssem.at[(_CPB - 1) & 1]).wait()

        @pl.when(bi + 2 < _NB)
        def _():
          start_stage(bi + 2)
        return 0

      lax.fori_loop(0, _NB, blk_body, 0)
      plsc.subcore_barrier()
      pltpu.sync_copy(acc.at[pl.ds(w * _RPT, _RPT)],
                      s_hbm.at[pl.ds(w * _RPT, _RPT), pl.ds(slab * _W32, _W32)])
      plsc.subcore_barrier()

    for slab in range(4):
      @pl.when(c == slab // 2)
      def _(slab=slab):
        slab_pass(xrefs[slab], slab)

  fn = pl.kernel(
      body,
      out_type=jax.ShapeDtypeStruct((_ROWS, _H), jnp.float32),
      mesh=mesh,
      compiler_params=pltpu.CompilerParams(use_tc_tiling_on_sc=False),
      scratch_types=[
          pltpu.VMEM_SHARED((_ROWS, _W32), jnp.float32),
          pltpu.VMEM((2, _BLK), jnp.int32),
          pltpu.VMEM((2, _BLK), jnp.int32),
          pltpu.VMEM((2, _BLK), jnp.int32),
          pltpu.VMEM((_CPB, _CH), jnp.int32),
          pltpu.VMEM((4, _CH, _W32), jnp.float32),
          pltpu.SemaphoreType.DMA((2,)),
          pltpu.SemaphoreType.DMA((4,)),
          pltpu.SemaphoreType.DMA((2,)),
          pltpu.SemaphoreType.DMA,
      ],
  )
  return fn(xs[0], xs[1], xs[2], xs[3], src, dst, typ,
            jnp.zeros((_RPT, _W32), jnp.float32))


def _lm_head(xflat, lm_W, lm_b, ln_g, ln_b):
  blk = 256
  grid = _N // blk

  def body(x_ref, w_ref, b_ref, g_ref, lb_ref, o_ref, s0, s1, s2, s3):
    h = jnp.dot(x_ref[...], w_ref[...], preferred_element_type=jnp.float32)
    h = jnp.maximum(h + b_ref[...], 0.0)
    mu = jnp.mean(h, axis=-1, keepdims=True)
    var = jnp.mean((h - mu) ** 2, axis=-1, keepdims=True)
    o = (h - mu) * lax.rsqrt(var + 1e-5) * g_ref[...] + lb_ref[...]
    o_ref[...] = o
    for s, ref in enumerate((s0, s1, s2, s3)):
      ref[...] = o[:, s * _W32:(s + 1) * _W32]

  xs_spec = pl.BlockSpec((blk, _W32), lambda i: (i, 0))
  xs_shape = jax.ShapeDtypeStruct((_N, _W32), jnp.float32)
  return pl.pallas_call(
      body,
      grid=(grid,),
      in_specs=[
          pl.BlockSpec((blk, _DLM), lambda i: (i, 0)),
          pl.BlockSpec((_DLM, _H), lambda i: (0, 0)),
          pl.BlockSpec((1, _H), lambda i: (0, 0)),
          pl.BlockSpec((1, _H), lambda i: (0, 0)),
          pl.BlockSpec((1, _H), lambda i: (0, 0)),
      ],
      out_specs=[pl.BlockSpec((blk, _H), lambda i: (i, 0))] + [xs_spec] * 4,
      out_shape=[jax.ShapeDtypeStruct((_N, _H), jnp.float32)] + [xs_shape] * 4,
  )(xflat, lm_W, lm_b.reshape(1, _H), ln_g.reshape(1, _H), ln_b.reshape(1, _H))


def _rgcn_dense(x, s4, inv3, root, wr, bias, emit_slabs):
  blk = 256
  grid = _N // blk

  def body(x_ref, s_ref, i_ref, r_ref, w_ref, b_ref, *outs):
    o = jnp.dot(x_ref[...], r_ref[...], preferred_element_type=jnp.float32)
    o = o + b_ref[...]
    for r in range(_R):
      iv1 = i_ref[r][:, 0:1]
      o = o + iv1 * jnp.dot(s_ref[r], w_ref[r],
                            preferred_element_type=jnp.float32)
    o = jnp.maximum(o, 0.0)
    outs[0][...] = o
    if emit_slabs:
      for s in range(4):
        outs[1 + s][...] = o[:, s * _W32:(s + 1) * _W32]

  out_specs = [pl.BlockSpec((blk, _H), lambda i: (i, 0))]
  out_shape = [jax.ShapeDtypeStruct((_N, _H), jnp.float32)]
  if emit_slabs:
    out_specs += [pl.BlockSpec((blk, _W32), lambda i: (i, 0))] * 4
    out_shape += [jax.ShapeDtypeStruct((_N, _W32), jnp.float32)] * 4
  return pl.pallas_call(
      body,
      grid=(grid,),
      in_specs=[
          pl.BlockSpec((blk, _H), lambda i: (i, 0)),
          pl.BlockSpec((_R, blk, _H), lambda i: (0, i, 0)),
          pl.BlockSpec((_R, blk, _W32), lambda i: (0, i, 0)),
          pl.BlockSpec((_H, _H), lambda i: (0, 0)),
          pl.BlockSpec((_R, _H, _H), lambda i: (0, 0, 0)),
          pl.BlockSpec((1, _H), lambda i: (0, 0)),
      ],
      out_specs=out_specs,
      out_shape=out_shape,
  )(x, s4, inv3, root, wr, bias.reshape(1, _H))


def _cls_head(lm, g, wpad, bpad, mask3d, labels3d):
  blk = 512
  grid = _N // blk

  def body(lm_ref, g_ref, w_ref, b_ref, m_ref, l_ref, lo_ref, loss_ref, acc):
    i = pl.program_id(0)
    logits = (jnp.dot(lm_ref[...], w_ref[0:_H], preferred_element_type=jnp.float32)
              + jnp.dot(g_ref[...], w_ref[_H:2 * _H], preferred_element_type=jnp.float32)
              + b_ref[...])
    lo_ref[...] = logits
    mx = jnp.max(logits, axis=-1, keepdims=True)
    lse = jnp.log(jnp.sum(jnp.exp(logits - mx), axis=-1, keepdims=True)) + mx
    lab = l_ref[0, 0, :]
    cols = lax.broadcasted_iota(jnp.int32, (blk, _H), 1)
    pick = jnp.sum(jnp.where(cols == lab[:, None], logits, 0.0), axis=-1,
                   keepdims=True)
    active = (m_ref[0, 0, :] == 1).astype(jnp.float32)[:, None]
    bsum = jnp.sum((lse - pick) * active)
    bcnt = jnp.sum(active)
    prev_s = jnp.where(i == 0, 0.0, acc[0])
    prev_c = jnp.where(i == 0, 0.0, acc[1])
    acc[0] = prev_s + bsum
    acc[1] = prev_c + bcnt

    @pl.when(i == grid - 1)
    def _():
      loss_ref[0, 0] = acc[0] / jnp.maximum(acc[1], 1.0)

  return pl.pallas_call(
      body,
      grid=(grid,),
      in_specs=[
          pl.BlockSpec((blk, _H), lambda i: (i, 0)),
          pl.BlockSpec((blk, _H), lambda i: (i, 0)),
          pl.BlockSpec((2 * _H, _H), lambda i: (0, 0)),
          pl.BlockSpec((1, _H), lambda i: (0, 0)),
          pl.BlockSpec((1, 1, blk), lambda i: (i, 0, 0)),
          pl.BlockSpec((1, 1, blk), lambda i: (i, 0, 0)),
      ],
      out_specs=[
          pl.BlockSpec((blk, _H), lambda i: (i, 0)),
          pl.BlockSpec((1, 1), lambda i: (0, 0), memory_space=pltpu.SMEM),
      ],
      out_shape=[
          jax.ShapeDtypeStruct((_N, _H), jnp.float32),
          jax.ShapeDtypeStruct((1, 1), jnp.float32),
      ],
      scratch_shapes=[pltpu.SMEM((2,), jnp.float32)],
  )(lm, g, wpad, bpad, mask3d, labels3d)


def kernel(output, edge_index, edge_type, attention_mask, labels,
           lm_W, lm_b, ln_g, ln_b, rgcn_W, rgcn_root, rgcn_bias, cls_W, cls_b):
  xflat = output.reshape(_N, _DLM)
  src = edge_index[0]
  dst = edge_index[1]

  inv = _sc_inv_counts(dst, edge_type)
  inv3 = inv.reshape(_R, _N, _W32)
  lm, a0, a1, a2, a3 = _lm_head(xflat, lm_W, lm_b, ln_g, ln_b)

  s0 = _sc_slab_sums((a0, a1, a2, a3), src, dst, edge_type)
  x1, b0, b1, b2, b3 = _rgcn_dense(lm, s0.reshape(_R, _N, _H), inv3,
                                   rgcn_root[0], rgcn_W[0], rgcn_bias[0],
                                   emit_slabs=True)
  s1 = _sc_slab_sums((b0, b1, b2, b3), src, dst, edge_type)
  (x2,) = _rgcn_dense(x1, s1.reshape(_R, _N, _H), inv3,
                      rgcn_root[1], rgcn_W[1], rgcn_bias[1], emit_slabs=False)

  wpad = jnp.zeros((2 * _H, _H), jnp.float32).at[:, :_C].set(cls_W)
  bpad = jnp.full((1, _H), -1e30, jnp.float32).at[0, :_C].set(cls_b)
  logits_pad, loss = _cls_head(lm, x2, wpad, bpad,
                               attention_mask.reshape(_N // 512, 1, 512),
                               labels.reshape(_N // 512, 1, 512))
  logits = logits_pad[:, :_C].reshape(_B, _SEQ, _C)
  return loss[0, 0], logits


# speedup vs baseline: 25.0078x; 1.0436x over previous
import jax
import jax.numpy as jnp
from jax import lax
from jax.experimental import pallas as pl
from jax.experimental.pallas import tpu as pltpu
from jax.experimental.pallas import tpu_sc as plsc

_B = 8
_SEQ = 2048
_DLM = 1024
_H = 128
_R = 3
_NL = 2
_C = 8
_N = _B * _SEQ
_E = 524288
_NT = 16
_W32 = 32
_ROWS = _R * _N
_EPT = _E // _NT
_BLK = 2048
_NB = _EPT // _BLK
_CH = 128
_CPB = _BLK // _CH
_RPT = _ROWS // _NT
_NCH = _EPT // _CH


def _memset_rows(ref, rows, value):
  def bd(i, _):
    ref[i, pl.ds(0, 16)] = jnp.full((16,), value, jnp.float32)
    ref[i, pl.ds(16, 16)] = jnp.full((16,), value, jnp.float32)
    return 0
  lax.fori_loop(0, rows, bd, 0)


def _sc_inv_counts(loc):
  mesh = plsc.VectorSubcoreMesh(core_axis_name="c", subcore_axis_name="s")

  def body(loc_hbm, zeros_hbm, inv_hbm, cnt, stage_loc,
           ones, ibuf, stgsem, csem, zsem):
    c = lax.axis_index("c")
    w = lax.axis_index("s")
    e0 = w * _EPT

    _memset_rows(ones, _CH, 1.0)

    pltpu.async_copy(zeros_hbm, cnt.at[pl.ds(w * _RPT, _RPT)], zsem).wait()
    plsc.subcore_barrier()

    r0 = w * (_EPT // _CH)

    def start_stage(bi):
      sb = bi & 1
      pltpu.async_copy(loc_hbm.at[pl.ds(r0 + bi * _CPB, _CPB)],
                       stage_loc.at[sb], stgsem.at[sb])

    def wait_stage(bi):
      sb = bi & 1
      pltpu.make_async_copy(loc_hbm.at[pl.ds(r0 + bi * _CPB, _CPB)],
                            stage_loc.at[sb], stgsem.at[sb]).wait()

    start_stage(0)
    start_stage(1)

    def blk_body(bi, _):
      sb = bi & 1
      wait_stage(bi)

      def sc_start(j, _):
        pltpu.async_copy(ones, cnt.at[stage_loc.at[sb, j]], csem, add=True)
        return 0
      lax.fori_loop(0, _CPB, sc_start, 0)

      def sc_wait(j, _):
        pltpu.make_async_copy(ones, cnt.at[stage_loc.at[sb, j]], csem).wait()
        return 0
      lax.fori_loop(0, _CPB, sc_wait, 0)

      @pl.when(bi + 2 < _NB)
      def _():
        start_stage(bi + 2)
      return 0

    lax.fori_loop(0, _NB, blk_body, 0)
    plsc.subcore_barrier()

    half = _ROWS // 2

    def inv_chunk(k, _):
      rg = c * half + w * (half // _NT) + k * _CH
      pltpu.sync_copy(cnt.at[pl.ds(rg, _CH)], ibuf)

      def inv_row(i, _):
        ibuf[i, pl.ds(0, 16)] = 1.0 / jnp.maximum(ibuf[i, pl.ds(0, 16)], 1.0)
        ibuf[i, pl.ds(16, 16)] = 1.0 / jnp.maximum(ibuf[i, pl.ds(16, 16)], 1.0)
        return 0
      lax.fori_loop(0, _CH, inv_row, 0)
      pltpu.sync_copy(ibuf, inv_hbm.at[pl.ds(rg, _CH)])
      return 0

    lax.fori_loop(0, (half // _NT) // _CH, inv_chunk, 0)

  fn = pl.kernel(
      body,
      out_type=jax.ShapeDtypeStruct((_ROWS, _W32), jnp.float32),
      mesh=mesh,
      compiler_params=pltpu.CompilerParams(use_tc_tiling_on_sc=False),
      scratch_types=[
          pltpu.VMEM_SHARED((_ROWS, _W32), jnp.float32),
          pltpu.VMEM((2, _CPB, _CH), jnp.int32),
          pltpu.VMEM((_CH, _W32), jnp.float32),
          pltpu.VMEM((_CH, _W32), jnp.float32),
          pltpu.SemaphoreType.DMA((2,)),
          pltpu.SemaphoreType.DMA,
          pltpu.SemaphoreType.DMA,
      ],
  )
  return fn(loc, jnp.zeros((_RPT, _W32), jnp.float32))


def _sc_slab_sums(xs, srcv, loc):
  mesh = plsc.VectorSubcoreMesh(core_axis_name="c", subcore_axis_name="s")

  def body(x0, x1, x2, x3, src_hbm, loc_hbm, zeros_hbm, s_hbm,
           acc, stage_s, stage_loc, gbuf, stgsem, gsem, ssem, zsem):
    c = lax.axis_index("c")
    w = lax.axis_index("s")
    e0 = w * _EPT
    r0 = w * (_EPT // _CH)
    xrefs = [x0, x1, x2, x3]

    def start_stage(b):
      sb = lax.rem(b, 3)
      pltpu.async_copy(src_hbm.at[pl.ds(e0 + b * _BLK, _BLK)],
                       stage_s.at[sb], stgsem.at[sb])
      pltpu.async_copy(loc_hbm.at[pl.ds(r0 + b * _CPB, _CPB)],
                       stage_loc.at[sb], stgsem.at[sb])

    def wait_stage(b):
      sb = lax.rem(b, 3)
      pltpu.make_async_copy(src_hbm.at[pl.ds(e0 + b * _BLK, _BLK)],
                            stage_s.at[sb], stgsem.at[sb]).wait()
      pltpu.make_async_copy(loc_hbm.at[pl.ds(r0 + b * _CPB, _CPB)],
                            stage_loc.at[sb], stgsem.at[sb]).wait()

    def slab_pass(xt, slab):
      pltpu.async_copy(zeros_hbm, acc.at[pl.ds(w * _RPT, _RPT)], zsem).wait()
      plsc.subcore_barrier()

      def gref(j):
        b = j >> 4
        return (xt.at[stage_s.at[lax.rem(b, 3), pl.ds((j & 15) * _CH, _CH)]],
                gbuf.at[j & 3], gsem.at[j & 3])

      def sref(j):
        b = j >> 4
        return (gbuf.at[j & 3], acc.at[stage_loc.at[lax.rem(b, 3), j & 15]],
                ssem.at[j & 1])

      start_stage(0)
      start_stage(1)
      wait_stage(0)
      for j in range(3):
        s_, d_, m_ = gref(j)
        pltpu.async_copy(s_, d_, m_)

      def chunk(j, _):
        b = j >> 4
        ch = j & 15

        @pl.when(jnp.logical_and(ch == 12, b + 1 < _NB))
        def _():
          wait_stage(b + 1)

        gs, gd, gm = gref(j)
        pltpu.make_async_copy(gs, gd, gm).wait()
        ss, sd, sm = sref(j)
        pltpu.async_copy(ss, sd, sm, add=True)

        @pl.when(j >= 1)
        def _():
          ps, pd, pm = sref(j - 1)
          pltpu.make_async_copy(ps, pd, pm).wait()

        @pl.when(j + 3 < _NCH)
        def _():
          ns, nd, nm = gref(j + 3)
          pltpu.async_copy(ns, nd, nm)

        @pl.when(jnp.logical_and(ch == 1, b + 2 < _NB))
        def _():
          start_stage(b + 2)
        return 0

      lax.fori_loop(0, _NCH, chunk, 0)
      ls, ld, lm = sref(_NCH - 1)
      pltpu.make_async_copy(ls, ld, lm).wait()
      plsc.subcore_barrier()
      pltpu.sync_copy(acc.at[pl.ds(w * _RPT, _RPT)],
                      s_hbm.at[pl.ds(w * _RPT, _RPT), pl.ds(slab * _W32, _W32)])
      plsc.subcore_barrier()

    for slab in range(4):
      @pl.when(c == slab // 2)
      def _(slab=slab):
        slab_pass(xrefs[slab], slab)

  fn = pl.kernel(
      body,
      out_type=jax.ShapeDtypeStruct((_ROWS, _H), jnp.float32),
      mesh=mesh,
      compiler_params=pltpu.CompilerParams(use_tc_tiling_on_sc=False),
      scratch_types=[
          pltpu.VMEM_SHARED((_ROWS, _W32), jnp.float32),
          pltpu.VMEM((3, _BLK), jnp.int32),
          pltpu.VMEM((3, _CPB, _CH), jnp.int32),
          pltpu.VMEM((4, _CH, _W32), jnp.float32),
          pltpu.SemaphoreType.DMA((3,)),
          pltpu.SemaphoreType.DMA((4,)),
          pltpu.SemaphoreType.DMA((2,)),
          pltpu.SemaphoreType.DMA,
      ],
  )
  return fn(xs[0], xs[1], xs[2], xs[3], srcv, loc,
            jnp.zeros((_RPT, _W32), jnp.float32))


def _loc_rows(dst2d, typ2d):
  grid = _E // (128 * 128)

  def body(d_ref, t_ref, o_ref):
    o_ref[...] = (t_ref[...] << 14) + d_ref[...]

  return pl.pallas_call(
      body,
      grid=(grid,),
      in_specs=[
          pl.BlockSpec((128, 128), lambda i: (i, 0)),
          pl.BlockSpec((128, 128), lambda i: (i, 0)),
      ],
      out_specs=pl.BlockSpec((128, 128), lambda i: (i, 0)),
      out_shape=jax.ShapeDtypeStruct((_E // 128, 128), jnp.int32),
  )(dst2d, typ2d)


def _lm_head(xflat, lm_W, lm_b, ln_g, ln_b):
  blk = 256
  grid = _N // blk

  def body(x_ref, w_ref, b_ref, g_ref, lb_ref, o_ref, s0, s1, s2, s3):
    h = jnp.dot(x_ref[...], w_ref[...], preferred_element_type=jnp.float32)
    h = jnp.maximum(h + b_ref[...], 0.0)
    mu = jnp.mean(h, axis=-1, keepdims=True)
    var = jnp.mean((h - mu) ** 2, axis=-1, keepdims=True)
    o = (h - mu) * lax.rsqrt(var + 1e-5) * g_ref[...] + lb_ref[...]
    o_ref[...] = o
    for s, ref in enumerate((s0, s1, s2, s3)):
      ref[...] = o[:, s * _W32:(s + 1) * _W32]

  xs_spec = pl.BlockSpec((blk, _W32), lambda i: (i, 0))
  xs_shape = jax.ShapeDtypeStruct((_N, _W32), jnp.float32)
  return pl.pallas_call(
      body,
      grid=(grid,),
      in_specs=[
          pl.BlockSpec((blk, _DLM), lambda i: (i, 0)),
          pl.BlockSpec((_DLM, _H), lambda i: (0, 0)),
          pl.BlockSpec((1, _H), lambda i: (0, 0)),
          pl.BlockSpec((1, _H), lambda i: (0, 0)),
          pl.BlockSpec((1, _H), lambda i: (0, 0)),
      ],
      out_specs=[pl.BlockSpec((blk, _H), lambda i: (i, 0))] + [xs_spec] * 4,
      out_shape=[jax.ShapeDtypeStruct((_N, _H), jnp.float32)] + [xs_shape] * 4,
  )(xflat, lm_W, lm_b.reshape(1, _H), ln_g.reshape(1, _H), ln_b.reshape(1, _H))


def _rgcn_dense(x, s4, inv3, root, wr, bias, emit_slabs):
  blk = 256
  grid = _N // blk

  def body(x_ref, s_ref, i_ref, r_ref, w_ref, b_ref, *outs):
    o = jnp.dot(x_ref[...], r_ref[...], preferred_element_type=jnp.float32)
    o = o + b_ref[...]
    for r in range(_R):
      iv1 = i_ref[r][:, 0:1]
      o = o + iv1 * jnp.dot(s_ref[r], w_ref[r],
                            preferred_element_type=jnp.float32)
    o = jnp.maximum(o, 0.0)
    outs[0][...] = o
    if emit_slabs:
      for s in range(4):
        outs[1 + s][...] = o[:, s * _W32:(s + 1) * _W32]

  out_specs = [pl.BlockSpec((blk, _H), lambda i: (i, 0))]
  out_shape = [jax.ShapeDtypeStruct((_N, _H), jnp.float32)]
  if emit_slabs:
    out_specs += [pl.BlockSpec((blk, _W32), lambda i: (i, 0))] * 4
    out_shape += [jax.ShapeDtypeStruct((_N, _W32), jnp.float32)] * 4
  return pl.pallas_call(
      body,
      grid=(grid,),
      in_specs=[
          pl.BlockSpec((blk, _H), lambda i: (i, 0)),
          pl.BlockSpec((_R, blk, _H), lambda i: (0, i, 0)),
          pl.BlockSpec((_R, blk, _W32), lambda i: (0, i, 0)),
          pl.BlockSpec((_H, _H), lambda i: (0, 0)),
          pl.BlockSpec((_R, _H, _H), lambda i: (0, 0, 0)),
          pl.BlockSpec((1, _H), lambda i: (0, 0)),
      ],
      out_specs=out_specs,
      out_shape=out_shape,
  )(x, s4, inv3, root, wr, bias.reshape(1, _H))


def _cls_head(lm, g, wpad, bpad, mask3d, labels3d):
  blk = 512
  grid = _N // blk

  def body(lm_ref, g_ref, w_ref, b_ref, m_ref, l_ref, lo_ref, loss_ref, acc):
    i = pl.program_id(0)
    logits = (jnp.dot(lm_ref[...], w_ref[0:_H], preferred_element_type=jnp.float32)
              + jnp.dot(g_ref[...], w_ref[_H:2 * _H], preferred_element_type=jnp.float32)
              + b_ref[...])
    lo_ref[...] = logits
    mx = jnp.max(logits, axis=-1, keepdims=True)
    lse = jnp.log(jnp.sum(jnp.exp(logits - mx), axis=-1, keepdims=True)) + mx
    lab = l_ref[0, 0, :]
    cols = lax.broadcasted_iota(jnp.int32, (blk, _H), 1)
    pick = jnp.sum(jnp.where(cols == lab[:, None], logits, 0.0), axis=-1,
                   keepdims=True)
    active = (m_ref[0, 0, :] == 1).astype(jnp.float32)[:, None]
    bsum = jnp.sum((lse - pick) * active)
    bcnt = jnp.sum(active)
    prev_s = jnp.where(i == 0, 0.0, acc[0])
    prev_c = jnp.where(i == 0, 0.0, acc[1])
    acc[0] = prev_s + bsum
    acc[1] = prev_c + bcnt

    @pl.when(i == grid - 1)
    def _():
      loss_ref[0, 0] = acc[0] / jnp.maximum(acc[1], 1.0)

  return pl.pallas_call(
      body,
      grid=(grid,),
      in_specs=[
          pl.BlockSpec((blk, _H), lambda i: (i, 0)),
          pl.BlockSpec((blk, _H), lambda i: (i, 0)),
          pl.BlockSpec((2 * _H, _H), lambda i: (0, 0)),
          pl.BlockSpec((1, _H), lambda i: (0, 0)),
          pl.BlockSpec((1, 1, blk), lambda i: (i, 0, 0)),
          pl.BlockSpec((1, 1, blk), lambda i: (i, 0, 0)),
      ],
      out_specs=[
          pl.BlockSpec((blk, _H), lambda i: (i, 0)),
          pl.BlockSpec((1, 1), lambda i: (0, 0), memory_space=pltpu.SMEM),
      ],
      out_shape=[
          jax.ShapeDtypeStruct((_N, _H), jnp.float32),
          jax.ShapeDtypeStruct((1, 1), jnp.float32),
      ],
      scratch_shapes=[pltpu.SMEM((2,), jnp.float32)],
  )(lm, g, wpad, bpad, mask3d, labels3d)


def kernel(output, edge_index, edge_type, attention_mask, labels,
           lm_W, lm_b, ln_g, ln_b, rgcn_W, rgcn_root, rgcn_bias, cls_W, cls_b):
  xflat = output.reshape(_N, _DLM)
  src = edge_index[0]
  dst = edge_index[1]

  loc = _loc_rows(dst.reshape(_E // 128, 128), edge_type.reshape(_E // 128, 128))
  inv = _sc_inv_counts(loc)
  inv3 = inv.reshape(_R, _N, _W32)
  lm, a0, a1, a2, a3 = _lm_head(xflat, lm_W, lm_b, ln_g, ln_b)

  s0 = _sc_slab_sums((a0, a1, a2, a3), src, loc)
  x1, b0, b1, b2, b3 = _rgcn_dense(lm, s0.reshape(_R, _N, _H), inv3,
                                   rgcn_root[0], rgcn_W[0], rgcn_bias[0],
                                   emit_slabs=True)
  s1 = _sc_slab_sums((b0, b1, b2, b3), src, loc)
  (x2,) = _rgcn_dense(x1, s1.reshape(_R, _N, _H), inv3,
                      rgcn_root[1], rgcn_W[1], rgcn_bias[1], emit_slabs=False)

  wpad = jnp.zeros((2 * _H, _H), jnp.float32).at[:, :_C].set(cls_W)
  bpad = jnp.full((1, _H), -1e30, jnp.float32).at[0, :_C].set(cls_b)
  logits_pad, loss = _cls_head(lm, x2, wpad, bpad,
                               attention_mask.reshape(_N // 512, 1, 512),
                               labels.reshape(_N // 512, 1, 512))
  logits = logits_pad[:, :_C].reshape(_B, _SEQ, _C)
  return loss[0, 0], logits


# speedup vs baseline: 27.5019x; 1.0997x over previous
import jax
import jax.numpy as jnp
from jax import lax
from jax.experimental import pallas as pl
from jax.experimental.pallas import tpu as pltpu
from jax.experimental.pallas import tpu_sc as plsc

_B = 8
_SEQ = 2048
_DLM = 1024
_H = 128
_R = 3
_NL = 2
_C = 8
_N = _B * _SEQ
_E = 524288
_NT = 16
_W32 = 32
_ROWS = _R * _N
_EPT = _E // _NT
_BLK = 2048
_NB = _EPT // _BLK
_CH = 128
_CPB = _BLK // _CH
_RPT = _ROWS // _NT
_NCH = _EPT // _CH


def _memset_rows(ref, rows, value):
  def bd(i, _):
    ref[i, pl.ds(0, 16)] = jnp.full((16,), value, jnp.float32)
    ref[i, pl.ds(16, 16)] = jnp.full((16,), value, jnp.float32)
    return 0
  lax.fori_loop(0, rows, bd, 0)


def _sc_inv_counts(loc):
  mesh = plsc.VectorSubcoreMesh(core_axis_name="c", subcore_axis_name="s")

  def body(loc_hbm, zeros_hbm, inv_hbm, cnt, stage_loc,
           ones, ibuf, stgsem, csem, zsem):
    c = lax.axis_index("c")
    w = lax.axis_index("s")
    e0 = w * _EPT

    _memset_rows(ones, _CH, 1.0)

    pltpu.async_copy(zeros_hbm, cnt.at[pl.ds(w * _RPT, _RPT)], zsem).wait()
    plsc.subcore_barrier()

    r0 = w * (_EPT // _CH)

    def start_stage(bi):
      sb = bi & 1
      pltpu.async_copy(loc_hbm.at[pl.ds(r0 + bi * _CPB, _CPB)],
                       stage_loc.at[sb], stgsem.at[sb])

    def wait_stage(bi):
      sb = bi & 1
      pltpu.make_async_copy(loc_hbm.at[pl.ds(r0 + bi * _CPB, _CPB)],
                            stage_loc.at[sb], stgsem.at[sb]).wait()

    start_stage(0)
    start_stage(1)

    def blk_body(bi, _):
      sb = bi & 1
      wait_stage(bi)

      def sc_start(j, _):
        pltpu.async_copy(ones, cnt.at[stage_loc.at[sb, j]], csem, add=True)
        return 0
      lax.fori_loop(0, _CPB, sc_start, 0)

      def sc_wait(j, _):
        pltpu.make_async_copy(ones, cnt.at[stage_loc.at[sb, j]], csem).wait()
        return 0
      lax.fori_loop(0, _CPB, sc_wait, 0)

      @pl.when(bi + 2 < _NB)
      def _():
        start_stage(bi + 2)
      return 0

    lax.fori_loop(0, _NB, blk_body, 0)
    plsc.subcore_barrier()

    half = _ROWS // 2

    def inv_chunk(k, _):
      rg = c * half + w * (half // _NT) + k * _CH
      pltpu.sync_copy(cnt.at[pl.ds(rg, _CH)], ibuf)

      def inv_row(i, _):
        ibuf[i, pl.ds(0, 16)] = 1.0 / jnp.maximum(ibuf[i, pl.ds(0, 16)], 1.0)
        ibuf[i, pl.ds(16, 16)] = 1.0 / jnp.maximum(ibuf[i, pl.ds(16, 16)], 1.0)
        return 0
      lax.fori_loop(0, _CH, inv_row, 0)
      pltpu.sync_copy(ibuf, inv_hbm.at[pl.ds(rg, _CH)])
      return 0

    lax.fori_loop(0, (half // _NT) // _CH, inv_chunk, 0)

  fn = pl.kernel(
      body,
      out_type=jax.ShapeDtypeStruct((_ROWS, _W32), jnp.float32),
      mesh=mesh,
      compiler_params=pltpu.CompilerParams(use_tc_tiling_on_sc=False),
      scratch_types=[
          pltpu.VMEM_SHARED((_ROWS, _W32), jnp.float32),
          pltpu.VMEM((2, _CPB, _CH), jnp.int32),
          pltpu.VMEM((_CH, _W32), jnp.float32),
          pltpu.VMEM((_CH, _W32), jnp.float32),
          pltpu.SemaphoreType.DMA((2,)),
          pltpu.SemaphoreType.DMA,
          pltpu.SemaphoreType.DMA,
      ],
  )
  return fn(loc, jnp.zeros((_RPT, _W32), jnp.float32))


def _sc_slab_sums(x4, s4s, loc):
  mesh = plsc.VectorSubcoreMesh(core_axis_name="c", subcore_axis_name="s")

  def body(x4_hbm, s40, s41, s42, s43, loc_hbm, zeros_hbm, s_hbm,
           acc, stage_s, stage_loc, gbuf, stgsem, gsem, ssem, zsem):
    c = lax.axis_index("c")
    w = lax.axis_index("s")
    r0 = w * (_EPT // _CH)
    srefs = [s40, s41, s42, s43]

    def start_stage(b, idx_hbm):
      sb = lax.rem(b, 3)
      pltpu.async_copy(idx_hbm.at[pl.ds(r0 + b * _CPB, _CPB)],
                       stage_s.at[sb], stgsem.at[sb])
      pltpu.async_copy(loc_hbm.at[pl.ds(r0 + b * _CPB, _CPB)],
                       stage_loc.at[sb], stgsem.at[sb])

    def wait_stage(b, idx_hbm):
      sb = lax.rem(b, 3)
      pltpu.make_async_copy(idx_hbm.at[pl.ds(r0 + b * _CPB, _CPB)],
                            stage_s.at[sb], stgsem.at[sb]).wait()
      pltpu.make_async_copy(loc_hbm.at[pl.ds(r0 + b * _CPB, _CPB)],
                            stage_loc.at[sb], stgsem.at[sb]).wait()

    def slab_pass(idx_hbm, slab):
      pltpu.async_copy(zeros_hbm, acc.at[pl.ds(w * _RPT, _RPT)], zsem).wait()
      plsc.subcore_barrier()

      def gref(j):
        b = j >> 4
        return (x4_hbm.at[stage_s.at[lax.rem(b, 3), j & 15]],
                gbuf.at[j & 3], gsem.at[j & 3])

      def sref(j):
        b = j >> 4
        return (gbuf.at[j & 3], acc.at[stage_loc.at[lax.rem(b, 3), j & 15]],
                ssem.at[j & 1])

      start_stage(0, idx_hbm)
      start_stage(1, idx_hbm)
      wait_stage(0, idx_hbm)
      for j in range(3):
        s_, d_, m_ = gref(j)
        pltpu.async_copy(s_, d_, m_)

      def chunk(j, _):
        b = j >> 4
        ch = j & 15

        @pl.when(jnp.logical_and(ch == 12, b + 1 < _NB))
        def _():
          wait_stage(b + 1, idx_hbm)

        gs, gd, gm = gref(j)
        pltpu.make_async_copy(gs, gd, gm).wait()
        ss, sd, sm = sref(j)
        pltpu.async_copy(ss, sd, sm, add=True)

        @pl.when(j >= 1)
        def _():
          ps, pd, pm = sref(j - 1)
          pltpu.make_async_copy(ps, pd, pm).wait()

        @pl.when(j + 3 < _NCH)
        def _():
          ns, nd, nm = gref(j + 3)
          pltpu.async_copy(ns, nd, nm)

        @pl.when(jnp.logical_and(ch == 1, b + 2 < _NB))
        def _():
          start_stage(b + 2, idx_hbm)
        return 0

      lax.fori_loop(0, _NCH, chunk, 0)
      ls, ld, lm = sref(_NCH - 1)
      pltpu.make_async_copy(ls, ld, lm).wait()
      plsc.subcore_barrier()
      pltpu.sync_copy(acc.at[pl.ds(w * _RPT, _RPT)],
                      s_hbm.at[pl.ds(w * _RPT, _RPT), pl.ds(slab * _W32, _W32)])
      plsc.subcore_barrier()

    for slab in range(4):
      @pl.when(c == slab // 2)
      def _(slab=slab):
        slab_pass(srefs[slab], slab)

  fn = pl.kernel(
      body,
      out_type=jax.ShapeDtypeStruct((_ROWS, _H), jnp.float32),
      mesh=mesh,
      compiler_params=pltpu.CompilerParams(use_tc_tiling_on_sc=False),
      scratch_types=[
          pltpu.VMEM_SHARED((_ROWS, _W32), jnp.float32),
          pltpu.VMEM((3, _CPB, _CH), jnp.int32),
          pltpu.VMEM((3, _CPB, _CH), jnp.int32),
          pltpu.VMEM((4, _CH, _W32), jnp.float32),
          pltpu.SemaphoreType.DMA((3,)),
          pltpu.SemaphoreType.DMA((4,)),
          pltpu.SemaphoreType.DMA((2,)),
          pltpu.SemaphoreType.DMA,
      ],
  )
  return fn(x4, s4s[0], s4s[1], s4s[2], s4s[3], loc,
            jnp.zeros((_RPT, _W32), jnp.float32))


def _edge_prep(dst2d, typ2d, src2d):
  grid = _E // (128 * 128)

  def body(d_ref, t_ref, s_ref, loc_o, *s4_os):
    loc_o[...] = (t_ref[...] << 14) + d_ref[...]
    s4 = s_ref[...] << 2
    for k in range(4):
      s4_os[k][...] = s4 + k

  spec = pl.BlockSpec((128, 128), lambda i: (i, 0))
  shape = jax.ShapeDtypeStruct((_E // 128, 128), jnp.int32)
  return pl.pallas_call(
      body,
      grid=(grid,),
      in_specs=[spec] * 3,
      out_specs=[spec] * 5,
      out_shape=[shape] * 5,
  )(dst2d, typ2d, src2d)


def _lm_head(xflat, lm_W, lm_b, ln_g, ln_b):
  blk = 256
  grid = _N // blk

  def body(x_ref, w_ref, b_ref, g_ref, lb_ref, o_ref):
    h = jnp.dot(x_ref[...], w_ref[...], preferred_element_type=jnp.float32)
    h = jnp.maximum(h + b_ref[...], 0.0)
    mu = jnp.mean(h, axis=-1, keepdims=True)
    var = jnp.mean((h - mu) ** 2, axis=-1, keepdims=True)
    o_ref[...] = (h - mu) * lax.rsqrt(var + 1e-5) * g_ref[...] + lb_ref[...]

  return pl.pallas_call(
      body,
      grid=(grid,),
      in_specs=[
          pl.BlockSpec((blk, _DLM), lambda i: (i, 0)),
          pl.BlockSpec((_DLM, _H), lambda i: (0, 0)),
          pl.BlockSpec((1, _H), lambda i: (0, 0)),
          pl.BlockSpec((1, _H), lambda i: (0, 0)),
          pl.BlockSpec((1, _H), lambda i: (0, 0)),
      ],
      out_specs=pl.BlockSpec((blk, _H), lambda i: (i, 0)),
      out_shape=jax.ShapeDtypeStruct((_N, _H), jnp.float32),
  )(xflat, lm_W, lm_b.reshape(1, _H), ln_g.reshape(1, _H), ln_b.reshape(1, _H))


def _rgcn_dense(x, s4, inv3, root, wr, bias):
  blk = 256
  grid = _N // blk

  def body(x_ref, s_ref, i_ref, r_ref, w_ref, b_ref, *outs):
    o = jnp.dot(x_ref[...], r_ref[...], preferred_element_type=jnp.float32)
    o = o + b_ref[...]
    for r in range(_R):
      iv1 = i_ref[r][:, 0:1]
      o = o + iv1 * jnp.dot(s_ref[r], w_ref[r],
                            preferred_element_type=jnp.float32)
    outs[0][...] = jnp.maximum(o, 0.0)

  out_specs = [pl.BlockSpec((blk, _H), lambda i: (i, 0))]
  out_shape = [jax.ShapeDtypeStruct((_N, _H), jnp.float32)]
  return pl.pallas_call(
      body,
      grid=(grid,),
      in_specs=[
          pl.BlockSpec((blk, _H), lambda i: (i, 0)),
          pl.BlockSpec((_R, blk, _H), lambda i: (0, i, 0)),
          pl.BlockSpec((_R, blk, _W32), lambda i: (0, i, 0)),
          pl.BlockSpec((_H, _H), lambda i: (0, 0)),
          pl.BlockSpec((_R, _H, _H), lambda i: (0, 0, 0)),
          pl.BlockSpec((1, _H), lambda i: (0, 0)),
      ],
      out_specs=out_specs,
      out_shape=out_shape,
  )(x, s4, inv3, root, wr, bias.reshape(1, _H))


def _cls_head(lm, g, wpad, bpad, mask3d, labels3d):
  blk = 512
  grid = _N // blk

  def body(lm_ref, g_ref, w_ref, b_ref, m_ref, l_ref, lo_ref, loss_ref, acc):
    i = pl.program_id(0)
    logits = (jnp.dot(lm_ref[...], w_ref[0:_H], preferred_element_type=jnp.float32)
              + jnp.dot(g_ref[...], w_ref[_H:2 * _H], preferred_element_type=jnp.float32)
              + b_ref[...])
    lo_ref[...] = logits
    mx = jnp.max(logits, axis=-1, keepdims=True)
    lse = jnp.log(jnp.sum(jnp.exp(logits - mx), axis=-1, keepdims=True)) + mx
    lab = l_ref[0, 0, :]
    cols = lax.broadcasted_iota(jnp.int32, (blk, _H), 1)
    pick = jnp.sum(jnp.where(cols == lab[:, None], logits, 0.0), axis=-1,
                   keepdims=True)
    active = (m_ref[0, 0, :] == 1).astype(jnp.float32)[:, None]
    bsum = jnp.sum((lse - pick) * active)
    bcnt = jnp.sum(active)
    prev_s = jnp.where(i == 0, 0.0, acc[0])
    prev_c = jnp.where(i == 0, 0.0, acc[1])
    acc[0] = prev_s + bsum
    acc[1] = prev_c + bcnt

    @pl.when(i == grid - 1)
    def _():
      loss_ref[0, 0] = acc[0] / jnp.maximum(acc[1], 1.0)

  return pl.pallas_call(
      body,
      grid=(grid,),
      in_specs=[
          pl.BlockSpec((blk, _H), lambda i: (i, 0)),
          pl.BlockSpec((blk, _H), lambda i: (i, 0)),
          pl.BlockSpec((2 * _H, _H), lambda i: (0, 0)),
          pl.BlockSpec((1, _H), lambda i: (0, 0)),
          pl.BlockSpec((1, 1, blk), lambda i: (i, 0, 0)),
          pl.BlockSpec((1, 1, blk), lambda i: (i, 0, 0)),
      ],
      out_specs=[
          pl.BlockSpec((blk, _H), lambda i: (i, 0)),
          pl.BlockSpec((1, 1), lambda i: (0, 0), memory_space=pltpu.SMEM),
      ],
      out_shape=[
          jax.ShapeDtypeStruct((_N, _H), jnp.float32),
          jax.ShapeDtypeStruct((1, 1), jnp.float32),
      ],
      scratch_shapes=[pltpu.SMEM((2,), jnp.float32)],
  )(lm, g, wpad, bpad, mask3d, labels3d)


def kernel(output, edge_index, edge_type, attention_mask, labels,
           lm_W, lm_b, ln_g, ln_b, rgcn_W, rgcn_root, rgcn_bias, cls_W, cls_b):
  xflat = output.reshape(_N, _DLM)
  src = edge_index[0]
  dst = edge_index[1]

  loc, s40, s41, s42, s43 = _edge_prep(dst.reshape(_E // 128, 128),
                                       edge_type.reshape(_E // 128, 128),
                                       src.reshape(_E // 128, 128))
  inv = _sc_inv_counts(loc)
  inv3 = inv.reshape(_R, _N, _W32)
  lm = _lm_head(xflat, lm_W, lm_b, ln_g, ln_b)

  s0 = _sc_slab_sums(lm.reshape(4 * _N, _W32), (s40, s41, s42, s43), loc)
  (x1,) = _rgcn_dense(lm, s0.reshape(_R, _N, _H), inv3,
                      rgcn_root[0], rgcn_W[0], rgcn_bias[0])
  s1 = _sc_slab_sums(x1.reshape(4 * _N, _W32), (s40, s41, s42, s43), loc)
  (x2,) = _rgcn_dense(x1, s1.reshape(_R, _N, _H), inv3,
                      rgcn_root[1], rgcn_W[1], rgcn_bias[1])

  wpad = jnp.zeros((2 * _H, _H), jnp.float32).at[:, :_C].set(cls_W)
  bpad = jnp.full((1, _H), -1e30, jnp.float32).at[0, :_C].set(cls_b)
  logits_pad, loss = _cls_head(lm, x2, wpad, bpad,
                               attention_mask.reshape(_N // 512, 1, 512),
                               labels.reshape(_N // 512, 1, 512))
  logits = logits_pad[:, :_C].reshape(_B, _SEQ, _C)
  return loss[0, 0], logits
